# Initial kernel scaffold; baseline (speedup 1.0000x reference)
import jax
import jax.numpy as jnp
from jax.experimental import pallas as pl


def kernel(x, edge_index, edge_attr, velocity_sequence_noise, params):
    raise NotImplementedError("write your pallas kernel here")


# speedup vs baseline: 1.1024x; 1.1024x over previous
import functools

import jax
import jax.numpy as jnp
from jax import lax
from jax.experimental import pallas as pl
from jax.experimental.pallas import tpu as pltpu

N_NODES = 10000
N_EDGES = 160000
H = 128

B_NODE = 1000
B_EDGE = 2000


def _ln(h, g, b):
    mu = jnp.mean(h, axis=-1, keepdims=True)
    var = jnp.mean((h - mu) * (h - mu), axis=-1, keepdims=True)
    return (h - mu) * lax.rsqrt(var + 1e-5) * g + b


def _dot(a, b):
    return jnp.dot(a, b, preferred_element_type=jnp.float32)


def _full(shape):
    return pl.BlockSpec(shape, lambda i: (0,) * len(shape))


def _enc_body(x_ref, w0, b0, w1, b1, w2, b2, g, bln, o_ref):
    h = jnp.maximum(_dot(x_ref[...], w0[...]) + b0[...], 0.0)
    h = jnp.maximum(_dot(h, w1[...]) + b1[...], 0.0)
    h = _dot(h, w2[...]) + b2[...]
    o_ref[...] = _ln(h, g[...], bln[...])


def _edge_body(g_ref, e_ref, we0, b0, w1, b1, w2, b2, g, bln,
               enew_ref, eout_ref):
    e = e_ref[...]
    h = jnp.maximum(g_ref[...] + _dot(e, we0[...]) + b0[...], 0.0)
    h = jnp.maximum(_dot(h, w1[...]) + b1[...], 0.0)
    h = _dot(h, w2[...]) + b2[...]
    enew = _ln(h, g[...], bln[...])
    enew_ref[...] = enew
    eout_ref[...] = e + enew


def _node_body(h_ref, a0_ref, a1_ref, w0h, w0a, b0, w1, b1, w2, b2, g, bln,
               hout_ref):
    hin = h_ref[...]
    a = a0_ref[...] + a1_ref[...]
    h = jnp.maximum(_dot(hin, w0h[...]) + _dot(a, w0a[...]) + b0[...], 0.0)
    h = jnp.maximum(_dot(h, w1[...]) + b1[...], 0.0)
    h = _dot(h, w2[...]) + b2[...]
    hout_ref[...] = hin + _ln(h, g[...], bln[...])


def _pre_body(h_ref, ws, wr, hs_ref, hr_ref):
    t = h_ref[...]
    hs_ref[...] = _dot(t, ws[...])
    hr_ref[...] = _dot(t, wr[...])


def _dec_body(h_ref, f_ref, w0, b0, w1, b1, w2, b2, std, mean, o_ref):
    h = jnp.maximum(_dot(h_ref[...], w0[...]) + b0[...], 0.0)
    h = jnp.maximum(_dot(h, w1[...]) + b1[...], 0.0)
    d = _dot(h, w2[...]) + b2[...]
    o_ref[...] = f_ref[...] + d * std[...] + mean[...]


def _row_spec(n_rows, b, k):
    return pl.BlockSpec((b, k), lambda i: (i, 0))


def _enc_call(x, w0, b0, w1, b1, w2, b2, g, bln, b_rows):
    n, k = x.shape
    grid = n // b_rows
    return pl.pallas_call(
        _enc_body,
        grid=(grid,),
        in_specs=[_row_spec(n, b_rows, k)] + [_full(w.shape) for w in
                  (w0, b0, w1, b1, w2, b2, g, bln)],
        out_specs=_row_spec(n, b_rows, H),
        out_shape=jax.ShapeDtypeStruct((n, H), jnp.float32),
    )(x, w0, b0, w1, b1, w2, b2, g, bln)


def _edge_call(gsum, e, we0, b0, w1, b1, w2, b2, g, bln):
    grid = N_EDGES // B_EDGE
    spec = _row_spec(N_EDGES, B_EDGE, H)
    return pl.pallas_call(
        _edge_body,
        grid=(grid,),
        in_specs=[spec, spec] + [_full(w.shape) for w in
                  (we0, b0, w1, b1, w2, b2, g, bln)],
        out_specs=[spec, spec],
        out_shape=[jax.ShapeDtypeStruct((N_EDGES, H), jnp.float32)] * 2,
    )(gsum, e, we0, b0, w1, b1, w2, b2, g, bln)


def _node_call(h, a0, a1, w0h, w0a, b0, w1, b1, w2, b2, g, bln):
    grid = N_NODES // B_NODE
    spec = _row_spec(N_NODES, B_NODE, H)
    return pl.pallas_call(
        _node_body,
        grid=(grid,),
        in_specs=[spec, spec, spec] + [_full(w.shape) for w in
                  (w0h, w0a, b0, w1, b1, w2, b2, g, bln)],
        out_specs=spec,
        out_shape=jax.ShapeDtypeStruct((N_NODES, H), jnp.float32),
    )(h, a0, a1, w0h, w0a, b0, w1, b1, w2, b2, g, bln)


def _pre_call(h, ws, wr):
    grid = N_NODES // B_NODE
    spec = _row_spec(N_NODES, B_NODE, H)
    return pl.pallas_call(
        _pre_body,
        grid=(grid,),
        in_specs=[spec, _full(ws.shape), _full(wr.shape)],
        out_specs=[spec, spec],
        out_shape=[jax.ShapeDtypeStruct((N_NODES, H), jnp.float32)] * 2,
    )(h, ws, wr)


def _dec_call(h, frames_p, w0, b0, w1, b1, w2, b2, std, mean):
    grid = N_NODES // B_NODE
    return pl.pallas_call(
        _dec_body,
        grid=(grid,),
        in_specs=[_row_spec(N_NODES, B_NODE, H),
                  _row_spec(N_NODES, B_NODE, 8)] +
                 [_full(w.shape) for w in (w0, b0, w1, b1, w2, b2, std, mean)],
        out_specs=_row_spec(N_NODES, B_NODE, 8),
        out_shape=jax.ShapeDtypeStruct((N_NODES, 8), jnp.float32),
    )(h, frames_p, w0, b0, w1, b1, w2, b2, std, mean)


def _gather_sum(hs, hr, senders, receivers):
    return hs[senders] + hr[receivers]


def _scatter_add(e_new, receivers):
    agg = jax.ops.segment_sum(e_new, receivers, num_segments=N_NODES)
    return agg, jnp.zeros_like(agg)


def _r2(b):
    return b.reshape(1, -1)


def kernel(x, edge_index, edge_attr, velocity_sequence_noise, params):
    del velocity_sequence_noise
    frames = x[:, 1:3]
    node_type = x[:, 0].astype(jnp.int32)
    one_hot = jax.nn.one_hot(node_type, 9, dtype=jnp.float32)
    node_feats = jnp.concatenate([frames, one_hot], axis=1)
    nn = params["node_norm"]
    node_attr = (node_feats - nn["mean"]) / nn["std"]
    node_attr_p = jnp.pad(node_attr, ((0, 0), (0, 5)))
    edge_attr_p = jnp.pad(edge_attr, ((0, 0), (0, 4)))

    enb, eeb = params["enc_nb"], params["enc_eb"]
    h = _enc_call(node_attr_p,
                  jnp.pad(enb["l0"]["W"], ((0, 5), (0, 0))), _r2(enb["l0"]["b"]),
                  enb["l1"]["W"], _r2(enb["l1"]["b"]),
                  enb["l2"]["W"], _r2(enb["l2"]["b"]),
                  _r2(enb["ln"]["g"]), _r2(enb["ln"]["b"]), B_NODE)
    e = _enc_call(edge_attr_p,
                  jnp.pad(eeb["l0"]["W"], ((0, 4), (0, 0))), _r2(eeb["l0"]["b"]),
                  eeb["l1"]["W"], _r2(eeb["l1"]["b"]),
                  eeb["l2"]["W"], _r2(eeb["l2"]["b"]),
                  _r2(eeb["ln"]["g"]), _r2(eeb["ln"]["b"]), B_EDGE)

    senders = edge_index[0]
    receivers = edge_index[1]

    for blk in params["blocks"]:
        eb, nb = blk["eb"], blk["nb"]
        w0 = eb["l0"]["W"]
        ws, wr, we = w0[:H], w0[H:2 * H], w0[2 * H:]
        hs, hr = _pre_call(h, ws, wr)
        gsum = _gather_sum(hs, hr, senders, receivers)
        e_new, e = _edge_call(gsum, e, we, _r2(eb["l0"]["b"]),
                              eb["l1"]["W"], _r2(eb["l1"]["b"]),
                              eb["l2"]["W"], _r2(eb["l2"]["b"]),
                              _r2(eb["ln"]["g"]), _r2(eb["ln"]["b"]))
        a0, a1 = _scatter_add(e_new, receivers)
        n0 = nb["l0"]["W"]
        h = _node_call(h, a0, a1, n0[:H], n0[H:], _r2(nb["l0"]["b"]),
                       nb["l1"]["W"], _r2(nb["l1"]["b"]),
                       nb["l2"]["W"], _r2(nb["l2"]["b"]),
                       _r2(nb["ln"]["g"]), _r2(nb["ln"]["b"]))

    dec = params["dec"]
    on = params["out_norm"]
    frames_p = jnp.pad(frames, ((0, 0), (0, 6)))
    w2p = jnp.pad(dec["l2"]["W"], ((0, 0), (0, 6)))
    b2p = jnp.pad(dec["l2"]["b"], (0, 6))
    stdp = jnp.pad(on["std"], (0, 6), constant_values=1.0)
    meanp = jnp.pad(on["mean"], (0, 6))
    out = _dec_call(h, frames_p,
                    dec["l0"]["W"], _r2(dec["l0"]["b"]),
                    dec["l1"]["W"], _r2(dec["l1"]["b"]),
                    w2p, _r2(b2p), _r2(stdp), _r2(meanp))
    return out[:, :2]


# speedup vs baseline: 3.0543x; 2.7707x over previous
import functools

import jax
import jax.numpy as jnp
from jax import lax
from jax.experimental import pallas as pl
from jax.experimental.pallas import tpu as pltpu
from jax.experimental.pallas import tpu_sc as plsc

N_NODES = 10000
N_EDGES = 160000
H = 128

B_NODE = 1000
B_EDGE = 2000

NC = 2
NS = 16
NW = NC * NS
BPW = N_EDGES // NW
CG = 128
NCH = BPW // CG
TAIL = BPW - NCH * CG
NPT = 632
NPT_LAST = N_NODES - (NS - 1) * NPT


def _ln(h, g, b):
    mu = jnp.mean(h, axis=-1, keepdims=True)
    var = jnp.mean((h - mu) * (h - mu), axis=-1, keepdims=True)
    return (h - mu) * lax.rsqrt(var + 1e-5) * g + b


def _dot(a, b):
    return jnp.dot(a, b, preferred_element_type=jnp.float32)


def _full(shape):
    return pl.BlockSpec(shape, lambda i: (0,) * len(shape))


def _enc_body(x_ref, w0, b0, w1, b1, w2, b2, g, bln, o_ref):
    h = jnp.maximum(_dot(x_ref[...], w0[...]) + b0[...], 0.0)
    h = jnp.maximum(_dot(h, w1[...]) + b1[...], 0.0)
    h = _dot(h, w2[...]) + b2[...]
    o_ref[...] = _ln(h, g[...], bln[...])


def _edge_body(g_ref, e_ref, we0, b0, w1, b1, w2, b2, g, bln,
               enew_ref, eout_ref):
    e = e_ref[...]
    h = jnp.maximum(g_ref[...] + _dot(e, we0[...]) + b0[...], 0.0)
    h = jnp.maximum(_dot(h, w1[...]) + b1[...], 0.0)
    h = _dot(h, w2[...]) + b2[...]
    enew = _ln(h, g[...], bln[...])
    enew_ref[...] = enew
    eout_ref[...] = e + enew


def _node_body(h_ref, a0_ref, a1_ref, w0h, w0a, b0, w1, b1, w2, b2, g, bln,
               hout_ref):
    hin = h_ref[...]
    a = a0_ref[...] + a1_ref[...]
    h = jnp.maximum(_dot(hin, w0h[...]) + _dot(a, w0a[...]) + b0[...], 0.0)
    h = jnp.maximum(_dot(h, w1[...]) + b1[...], 0.0)
    h = _dot(h, w2[...]) + b2[...]
    hout_ref[...] = hin + _ln(h, g[...], bln[...])


def _pre_body(h_ref, ws, wr, hs_ref, hr_ref):
    t = h_ref[...]
    hs_ref[...] = _dot(t, ws[...])
    hr_ref[...] = _dot(t, wr[...])


def _dec_body(h_ref, f_ref, w0, b0, w1, b1, w2, b2, std, mean, o_ref):
    h = jnp.maximum(_dot(h_ref[...], w0[...]) + b0[...], 0.0)
    h = jnp.maximum(_dot(h, w1[...]) + b1[...], 0.0)
    d = _dot(h, w2[...]) + b2[...]
    o_ref[...] = f_ref[...] + d * std[...] + mean[...]


def _row_spec(n_rows, b, k):
    return pl.BlockSpec((b, k), lambda i: (i, 0))


def _enc_call(x, w0, b0, w1, b1, w2, b2, g, bln, b_rows):
    n, k = x.shape
    grid = n // b_rows
    return pl.pallas_call(
        _enc_body,
        grid=(grid,),
        in_specs=[_row_spec(n, b_rows, k)] + [_full(w.shape) for w in
                  (w0, b0, w1, b1, w2, b2, g, bln)],
        out_specs=_row_spec(n, b_rows, H),
        out_shape=jax.ShapeDtypeStruct((n, H), jnp.float32),
    )(x, w0, b0, w1, b1, w2, b2, g, bln)


def _edge_call(gsum, e, we0, b0, w1, b1, w2, b2, g, bln):
    grid = N_EDGES // B_EDGE
    spec = _row_spec(N_EDGES, B_EDGE, H)
    return pl.pallas_call(
        _edge_body,
        grid=(grid,),
        in_specs=[spec, spec] + [_full(w.shape) for w in
                  (we0, b0, w1, b1, w2, b2, g, bln)],
        out_specs=[spec, spec],
        out_shape=[jax.ShapeDtypeStruct((N_EDGES, H), jnp.float32)] * 2,
    )(gsum, e, we0, b0, w1, b1, w2, b2, g, bln)


def _node_call(h, a0, a1, w0h, w0a, b0, w1, b1, w2, b2, g, bln):
    grid = N_NODES // B_NODE
    spec = _row_spec(N_NODES, B_NODE, H)
    return pl.pallas_call(
        _node_body,
        grid=(grid,),
        in_specs=[spec, spec, spec] + [_full(w.shape) for w in
                  (w0h, w0a, b0, w1, b1, w2, b2, g, bln)],
        out_specs=spec,
        out_shape=jax.ShapeDtypeStruct((N_NODES, H), jnp.float32),
    )(h, a0, a1, w0h, w0a, b0, w1, b1, w2, b2, g, bln)


def _pre_call(h, ws, wr):
    grid = N_NODES // B_NODE
    spec = _row_spec(N_NODES, B_NODE, H)
    return pl.pallas_call(
        _pre_body,
        grid=(grid,),
        in_specs=[spec, _full(ws.shape), _full(wr.shape)],
        out_specs=[spec, spec],
        out_shape=[jax.ShapeDtypeStruct((N_NODES, H), jnp.float32)] * 2,
    )(h, ws, wr)


def _dec_call(h, frames_p, w0, b0, w1, b1, w2, b2, std, mean):
    grid = N_NODES // B_NODE
    return pl.pallas_call(
        _dec_body,
        grid=(grid,),
        in_specs=[_row_spec(N_NODES, B_NODE, H),
                  _row_spec(N_NODES, B_NODE, 8)] +
                 [_full(w.shape) for w in (w0, b0, w1, b1, w2, b2, std, mean)],
        out_specs=_row_spec(N_NODES, B_NODE, 8),
        out_shape=jax.ShapeDtypeStruct((N_NODES, 8), jnp.float32),
    )(h, frames_p, w0, b0, w1, b1, w2, b2, std, mean)


def _sc_mesh():
    return plsc.VectorSubcoreMesh(core_axis_name="c", subcore_axis_name="s",
                                  num_cores=NC, num_subcores=NS)


@functools.cache
def _gather_sum_kernel():
    @functools.partial(
        pl.kernel,
        out_type=jax.ShapeDtypeStruct((N_EDGES, H), jnp.float32),
        mesh=_sc_mesh(),
        scratch_types=[
            pltpu.VMEM((BPW,), jnp.int32),
            pltpu.VMEM((BPW,), jnp.int32),
            pltpu.VMEM((CG, H), jnp.float32),
            pltpu.VMEM((CG, H), jnp.float32),
            pltpu.SemaphoreType.DMA,
            pltpu.SemaphoreType.DMA,
        ],
    )
    def gather_sum(hs_hbm, hr_hbm, s_hbm, r_hbm, out_hbm,
                   sidx, ridx, srows, rrows, sem1, sem2):
        cid = lax.axis_index("c")
        sid = lax.axis_index("s")
        base = (sid * NC + cid) * BPW
        pltpu.sync_copy(s_hbm.at[pl.ds(base, BPW)], sidx)
        pltpu.sync_copy(r_hbm.at[pl.ds(base, BPW)], ridx)

        def do_chunk(off, n):
            cp1 = pltpu.async_copy(hs_hbm.at[sidx.at[pl.ds(off, n)]],
                                   srows.at[pl.ds(0, n)], sem1)
            cp2 = pltpu.async_copy(hr_hbm.at[ridx.at[pl.ds(off, n)]],
                                   rrows.at[pl.ds(0, n)], sem2)
            cp1.wait()
            cp2.wait()

            def row(i, c):
                for j in range(H // 16):
                    sl = pl.ds(j * 16, 16)
                    srows[i, sl] = srows[i, sl] + rrows[i, sl]
                return c
            lax.fori_loop(0, n, row, 0)
            pltpu.sync_copy(srows.at[pl.ds(0, n)],
                            out_hbm.at[pl.ds(base + off, n)])

        def chunk(ci, c):
            do_chunk(ci * CG, CG)
            return c
        lax.fori_loop(0, NCH, chunk, 0)
        do_chunk(NCH * CG, TAIL)

    return gather_sum


@functools.cache
def _scatter_add_kernel():
    @functools.partial(
        pl.kernel,
        out_type=(jax.ShapeDtypeStruct((N_NODES, H), jnp.float32),
                  jax.ShapeDtypeStruct((N_NODES, H), jnp.float32)),
        mesh=_sc_mesh(),
        scratch_types=[
            pltpu.VMEM_SHARED((N_NODES, H), jnp.float32),
            pltpu.VMEM((CG,), jnp.int32),
            pltpu.VMEM((TAIL,), jnp.int32),
            pltpu.VMEM((CG, H), jnp.float32),
        ],
    )
    def scatter_add(enew_hbm, r_hbm, zeros_hbm, out0, out1,
                    acc, idxc, idxt, rows):
        cid = lax.axis_index("c")
        sid = lax.axis_index("s")
        nsl = pl.ds(sid * NPT, NPT)
        nsl_last = pl.ds((NS - 1) * NPT, NPT_LAST)

        @pl.when(sid < NS - 1)
        def _():
            pltpu.sync_copy(zeros_hbm.at[nsl], acc.at[nsl])

        @pl.when(sid == NS - 1)
        def _():
            pltpu.sync_copy(zeros_hbm.at[nsl_last], acc.at[nsl_last])
        plsc.subcore_barrier()

        base = cid * (N_EDGES // NC) + sid * BPW

        def do_chunk(off, n, idxbuf):
            pltpu.sync_copy(r_hbm.at[pl.ds(base + off, n)], idxbuf)
            pltpu.sync_copy(enew_hbm.at[pl.ds(base + off, n)],
                            rows.at[pl.ds(0, n)])
            pltpu.sync_copy(rows.at[pl.ds(0, n)], acc.at[idxbuf], add=True)

        def chunk(ci, c):
            do_chunk(ci * CG, CG, idxc)
            return c
        lax.fori_loop(0, NCH, chunk, 0)
        do_chunk(NCH * CG, TAIL, idxt)
        plsc.subcore_barrier()

        @pl.when((cid == 0) & (sid < NS - 1))
        def _():
            pltpu.sync_copy(acc.at[nsl], out0.at[nsl])

        @pl.when((cid == 0) & (sid == NS - 1))
        def _():
            pltpu.sync_copy(acc.at[nsl_last], out0.at[nsl_last])

        @pl.when((cid == 1) & (sid < NS - 1))
        def _():
            pltpu.sync_copy(acc.at[nsl], out1.at[nsl])

        @pl.when((cid == 1) & (sid == NS - 1))
        def _():
            pltpu.sync_copy(acc.at[nsl_last], out1.at[nsl_last])

    return scatter_add


def _gather_sum(hs, hr, senders, receivers):
    return _gather_sum_kernel()(hs, hr, senders, receivers)


def _scatter_add(e_new, receivers, zeros):
    return _scatter_add_kernel()(e_new, receivers, zeros)


def _r2(b):
    return b.reshape(1, -1)


def kernel(x, edge_index, edge_attr, velocity_sequence_noise, params):
    del velocity_sequence_noise
    frames = x[:, 1:3]
    node_type = x[:, 0].astype(jnp.int32)
    one_hot = jax.nn.one_hot(node_type, 9, dtype=jnp.float32)
    node_feats = jnp.concatenate([frames, one_hot], axis=1)
    nn = params["node_norm"]
    node_attr = (node_feats - nn["mean"]) / nn["std"]
    node_attr_p = jnp.pad(node_attr, ((0, 0), (0, 5)))
    edge_attr_p = jnp.pad(edge_attr, ((0, 0), (0, 4)))

    enb, eeb = params["enc_nb"], params["enc_eb"]
    h = _enc_call(node_attr_p,
                  jnp.pad(enb["l0"]["W"], ((0, 5), (0, 0))), _r2(enb["l0"]["b"]),
                  enb["l1"]["W"], _r2(enb["l1"]["b"]),
                  enb["l2"]["W"], _r2(enb["l2"]["b"]),
                  _r2(enb["ln"]["g"]), _r2(enb["ln"]["b"]), B_NODE)
    e = _enc_call(edge_attr_p,
                  jnp.pad(eeb["l0"]["W"], ((0, 4), (0, 0))), _r2(eeb["l0"]["b"]),
                  eeb["l1"]["W"], _r2(eeb["l1"]["b"]),
                  eeb["l2"]["W"], _r2(eeb["l2"]["b"]),
                  _r2(eeb["ln"]["g"]), _r2(eeb["ln"]["b"]), B_EDGE)

    senders = edge_index[0]
    receivers = edge_index[1]
    zeros = jnp.zeros((N_NODES, H), jnp.float32)

    for blk in params["blocks"]:
        eb, nb = blk["eb"], blk["nb"]
        w0 = eb["l0"]["W"]
        ws, wr, we = w0[:H], w0[H:2 * H], w0[2 * H:]
        hs, hr = _pre_call(h, ws, wr)
        gsum = _gather_sum(hs, hr, senders, receivers)
        e_new, e = _edge_call(gsum, e, we, _r2(eb["l0"]["b"]),
                              eb["l1"]["W"], _r2(eb["l1"]["b"]),
                              eb["l2"]["W"], _r2(eb["l2"]["b"]),
                              _r2(eb["ln"]["g"]), _r2(eb["ln"]["b"]))
        a0, a1 = _scatter_add(e_new, receivers, zeros)
        n0 = nb["l0"]["W"]
        h = _node_call(h, a0, a1, n0[:H], n0[H:], _r2(nb["l0"]["b"]),
                       nb["l1"]["W"], _r2(nb["l1"]["b"]),
                       nb["l2"]["W"], _r2(nb["l2"]["b"]),
                       _r2(nb["ln"]["g"]), _r2(nb["ln"]["b"]))

    dec = params["dec"]
    on = params["out_norm"]
    frames_p = jnp.pad(frames, ((0, 0), (0, 6)))
    w2p = jnp.pad(dec["l2"]["W"], ((0, 0), (0, 6)))
    b2p = jnp.pad(dec["l2"]["b"], (0, 6))
    stdp = jnp.pad(on["std"], (0, 6), constant_values=1.0)
    meanp = jnp.pad(on["mean"], (0, 6))
    out = _dec_call(h, frames_p,
                    dec["l0"]["W"], _r2(dec["l0"]["b"]),
                    dec["l1"]["W"], _r2(dec["l1"]["b"]),
                    w2p, _r2(b2p), _r2(stdp), _r2(meanp))
    return out[:, :2]


# speedup vs baseline: 3.9351x; 1.2884x over previous
import functools

import jax
import jax.numpy as jnp
from jax import lax
from jax.experimental import pallas as pl
from jax.experimental.pallas import tpu as pltpu
from jax.experimental.pallas import tpu_sc as plsc

N_NODES = 10000
N_EDGES = 160000
H = 128

B_NODE = 1000
B_EDGE = 2000

NC = 2
NS = 16
NW = NC * NS
BPW = N_EDGES // NW
CG = 128
NCH = BPW // CG
TAIL = BPW - NCH * CG
NPT = 632
NPT_LAST = N_NODES - (NS - 1) * NPT


def _ln(h, g, b):
    mu = jnp.mean(h, axis=-1, keepdims=True)
    var = jnp.mean((h - mu) * (h - mu), axis=-1, keepdims=True)
    return (h - mu) * lax.rsqrt(var + 1e-5) * g + b


def _dot(a, b):
    return jnp.dot(a, b, preferred_element_type=jnp.float32)


def _full(shape):
    return pl.BlockSpec(shape, lambda i: (0,) * len(shape))


def _enc_body(x_ref, w0, b0, w1, b1, w2, b2, g, bln, o_ref):
    h = jnp.maximum(_dot(x_ref[...], w0[...]) + b0[...], 0.0)
    h = jnp.maximum(_dot(h, w1[...]) + b1[...], 0.0)
    h = _dot(h, w2[...]) + b2[...]
    o_ref[...] = _ln(h, g[...], bln[...])


def _edge_body(g_ref, e_ref, we0, b0, w1, b1, w2, b2, g, bln,
               enew_ref, eout_ref):
    e = e_ref[...]
    h = jnp.maximum(g_ref[...] + _dot(e, we0[...]) + b0[...], 0.0)
    h = jnp.maximum(_dot(h, w1[...]) + b1[...], 0.0)
    h = _dot(h, w2[...]) + b2[...]
    enew = _ln(h, g[...], bln[...])
    enew_ref[...] = enew
    eout_ref[...] = e + enew


def _node_body(h_ref, a0_ref, a1_ref, w0h, w0a, b0, w1, b1, w2, b2, g, bln,
               hout_ref):
    hin = h_ref[...]
    a = a0_ref[...] + a1_ref[...]
    h = jnp.maximum(_dot(hin, w0h[...]) + _dot(a, w0a[...]) + b0[...], 0.0)
    h = jnp.maximum(_dot(h, w1[...]) + b1[...], 0.0)
    h = _dot(h, w2[...]) + b2[...]
    hout_ref[...] = hin + _ln(h, g[...], bln[...])


def _pre_body(h_ref, ws, wr, hs_ref, hr_ref):
    t = h_ref[...]
    hs_ref[...] = _dot(t, ws[...])
    hr_ref[...] = _dot(t, wr[...])


def _dec_body(h_ref, f_ref, w0, b0, w1, b1, w2, b2, std, mean, o_ref):
    h = jnp.maximum(_dot(h_ref[...], w0[...]) + b0[...], 0.0)
    h = jnp.maximum(_dot(h, w1[...]) + b1[...], 0.0)
    d = _dot(h, w2[...]) + b2[...]
    o_ref[...] = f_ref[...] + d * std[...] + mean[...]


def _row_spec(n_rows, b, k):
    return pl.BlockSpec((b, k), lambda i: (i, 0))


def _enc_call(x, w0, b0, w1, b1, w2, b2, g, bln, b_rows):
    n, k = x.shape
    grid = n // b_rows
    return pl.pallas_call(
        _enc_body,
        grid=(grid,),
        in_specs=[_row_spec(n, b_rows, k)] + [_full(w.shape) for w in
                  (w0, b0, w1, b1, w2, b2, g, bln)],
        out_specs=_row_spec(n, b_rows, H),
        out_shape=jax.ShapeDtypeStruct((n, H), jnp.float32),
    )(x, w0, b0, w1, b1, w2, b2, g, bln)


def _edge_call(gsum, e, we0, b0, w1, b1, w2, b2, g, bln):
    grid = N_EDGES // B_EDGE
    spec = _row_spec(N_EDGES, B_EDGE, H)
    return pl.pallas_call(
        _edge_body,
        grid=(grid,),
        in_specs=[spec, spec] + [_full(w.shape) for w in
                  (we0, b0, w1, b1, w2, b2, g, bln)],
        out_specs=[spec, spec],
        out_shape=[jax.ShapeDtypeStruct((N_EDGES, H), jnp.float32)] * 2,
    )(gsum, e, we0, b0, w1, b1, w2, b2, g, bln)


def _node_call(h, a0, a1, w0h, w0a, b0, w1, b1, w2, b2, g, bln):
    grid = N_NODES // B_NODE
    spec = _row_spec(N_NODES, B_NODE, H)
    return pl.pallas_call(
        _node_body,
        grid=(grid,),
        in_specs=[spec, spec, spec] + [_full(w.shape) for w in
                  (w0h, w0a, b0, w1, b1, w2, b2, g, bln)],
        out_specs=spec,
        out_shape=jax.ShapeDtypeStruct((N_NODES, H), jnp.float32),
    )(h, a0, a1, w0h, w0a, b0, w1, b1, w2, b2, g, bln)


def _pre_call(h, ws, wr):
    grid = N_NODES // B_NODE
    spec = _row_spec(N_NODES, B_NODE, H)
    return pl.pallas_call(
        _pre_body,
        grid=(grid,),
        in_specs=[spec, _full(ws.shape), _full(wr.shape)],
        out_specs=[spec, spec],
        out_shape=[jax.ShapeDtypeStruct((N_NODES, H), jnp.float32)] * 2,
    )(h, ws, wr)


def _dec_call(h, frames_p, w0, b0, w1, b1, w2, b2, std, mean):
    grid = N_NODES // B_NODE
    return pl.pallas_call(
        _dec_body,
        grid=(grid,),
        in_specs=[_row_spec(N_NODES, B_NODE, H),
                  _row_spec(N_NODES, B_NODE, 8)] +
                 [_full(w.shape) for w in (w0, b0, w1, b1, w2, b2, std, mean)],
        out_specs=_row_spec(N_NODES, B_NODE, 8),
        out_shape=jax.ShapeDtypeStruct((N_NODES, 8), jnp.float32),
    )(h, frames_p, w0, b0, w1, b1, w2, b2, std, mean)


def _sc_mesh():
    return plsc.VectorSubcoreMesh(core_axis_name="c", subcore_axis_name="s",
                                  num_cores=NC, num_subcores=NS)


@functools.cache
def _gather_sum_kernel():
    @functools.partial(
        pl.kernel,
        out_type=jax.ShapeDtypeStruct((N_EDGES, H), jnp.float32),
        mesh=_sc_mesh(),
        scratch_types=[
            pltpu.VMEM((BPW,), jnp.int32),
            pltpu.VMEM((BPW,), jnp.int32),
            pltpu.VMEM((CG, H), jnp.float32),
            pltpu.VMEM((CG, H), jnp.float32),
            pltpu.VMEM((CG, H), jnp.float32),
            pltpu.VMEM((CG, H), jnp.float32),
            pltpu.SemaphoreType.DMA,
            pltpu.SemaphoreType.DMA,
            pltpu.SemaphoreType.DMA,
            pltpu.SemaphoreType.DMA,
        ],
    )
    def gather_sum(hs_hbm, hr_hbm, s_hbm, r_hbm, out_hbm,
                   sidx, ridx, srows0, rrows0, srows1, rrows1,
                   ss0, sr0, ss1, sr1):
        cid = lax.axis_index("c")
        sid = lax.axis_index("s")
        base = (sid * NC + cid) * BPW
        pltpu.sync_copy(s_hbm.at[pl.ds(base, BPW)], sidx)
        pltpu.sync_copy(r_hbm.at[pl.ds(base, BPW)], ridx)
        bufs = ((srows0, rrows0, ss0, sr0), (srows1, rrows1, ss1, sr1))

        def issue(off, b):
            sb, rb, ss, sr = bufs[b]
            pltpu.async_copy(hs_hbm.at[sidx.at[pl.ds(off, CG)]], sb, ss)
            pltpu.async_copy(hr_hbm.at[ridx.at[pl.ds(off, CG)]], rb, sr)

        def drain_compute(off, b):
            sb, rb, ss, sr = bufs[b]
            pltpu.make_async_copy(hs_hbm.at[sidx.at[pl.ds(off, CG)]], sb,
                                  ss).wait()
            pltpu.make_async_copy(hr_hbm.at[ridx.at[pl.ds(off, CG)]], rb,
                                  sr).wait()

            def row(i, c):
                for j in range(H // 16):
                    sl = pl.ds(j * 16, 16)
                    sb[i, sl] = sb[i, sl] + rb[i, sl]
                return c
            lax.fori_loop(0, CG, row, 0)
            pltpu.sync_copy(sb, out_hbm.at[pl.ds(base + off, CG)])

        issue(0, 0)
        issue(CG, 1)

        def round_(g, c):
            k = g * 2
            drain_compute(k * CG, 0)

            @pl.when(k + 2 < NCH)
            def _():
                issue((k + 2) * CG, 0)
            drain_compute((k + 1) * CG, 1)

            @pl.when(k + 3 < NCH)
            def _():
                issue((k + 3) * CG, 1)
            return c
        lax.fori_loop(0, (NCH - 1) // 2, round_, 0)
        drain_compute((NCH - 1) * CG, 0)

        sb, rb, ss, _ = bufs[1]
        toff = NCH * CG
        pltpu.async_copy(hs_hbm.at[sidx.at[pl.ds(toff, TAIL)]],
                         sb.at[pl.ds(0, TAIL)], ss).wait()
        pltpu.async_copy(hr_hbm.at[ridx.at[pl.ds(toff, TAIL)]],
                         rb.at[pl.ds(0, TAIL)], ss).wait()

        def trow(i, c):
            for j in range(H // 16):
                sl = pl.ds(j * 16, 16)
                sb[i, sl] = sb[i, sl] + rb[i, sl]
            return c
        lax.fori_loop(0, TAIL, trow, 0)
        pltpu.sync_copy(sb.at[pl.ds(0, TAIL)],
                        out_hbm.at[pl.ds(base + toff, TAIL)])

    return gather_sum


@functools.cache
def _scatter_add_kernel():
    @functools.partial(
        pl.kernel,
        out_type=(jax.ShapeDtypeStruct((N_NODES, H), jnp.float32),
                  jax.ShapeDtypeStruct((N_NODES, H), jnp.float32)),
        mesh=_sc_mesh(),
        scratch_types=[
            pltpu.VMEM_SHARED((N_NODES, H), jnp.float32),
            pltpu.VMEM((CG,), jnp.int32),
            pltpu.VMEM((CG,), jnp.int32),
            pltpu.VMEM((TAIL,), jnp.int32),
            pltpu.VMEM((CG, H), jnp.float32),
            pltpu.VMEM((CG, H), jnp.float32),
            pltpu.SemaphoreType.DMA,
            pltpu.SemaphoreType.DMA,
            pltpu.SemaphoreType.DMA,
            pltpu.SemaphoreType.DMA,
        ],
    )
    def scatter_add(enew_hbm, r_hbm, zeros_hbm, out0, out1,
                    acc, idxc0, idxc1, idxt, rows0, rows1,
                    si0, sd0, si1, sd1):
        cid = lax.axis_index("c")
        sid = lax.axis_index("s")
        nsl = pl.ds(sid * NPT, NPT)
        nsl_last = pl.ds((NS - 1) * NPT, NPT_LAST)

        @pl.when(sid < NS - 1)
        def _():
            pltpu.sync_copy(zeros_hbm.at[nsl], acc.at[nsl])

        @pl.when(sid == NS - 1)
        def _():
            pltpu.sync_copy(zeros_hbm.at[nsl_last], acc.at[nsl_last])
        plsc.subcore_barrier()

        base = cid * (N_EDGES // NC) + sid * BPW
        bufs = ((idxc0, rows0, si0, sd0), (idxc1, rows1, si1, sd1))

        def issue(off, b):
            ib, rb, si, sd = bufs[b]
            pltpu.async_copy(r_hbm.at[pl.ds(base + off, CG)], ib, si)
            pltpu.async_copy(enew_hbm.at[pl.ds(base + off, CG)], rb, sd)

        def drain_scatter(off, b):
            ib, rb, si, sd = bufs[b]
            pltpu.make_async_copy(r_hbm.at[pl.ds(base + off, CG)], ib,
                                  si).wait()
            pltpu.make_async_copy(enew_hbm.at[pl.ds(base + off, CG)], rb,
                                  sd).wait()
            pltpu.sync_copy(rb, acc.at[ib], add=True)

        issue(0, 0)
        issue(CG, 1)

        def round_(g, c):
            k = g * 2
            drain_scatter(k * CG, 0)

            @pl.when(k + 2 < NCH)
            def _():
                issue((k + 2) * CG, 0)
            drain_scatter((k + 1) * CG, 1)

            @pl.when(k + 3 < NCH)
            def _():
                issue((k + 3) * CG, 1)
            return c
        lax.fori_loop(0, (NCH - 1) // 2, round_, 0)
        drain_scatter((NCH - 1) * CG, 0)

        toff = NCH * CG
        pltpu.sync_copy(r_hbm.at[pl.ds(base + toff, TAIL)], idxt)
        pltpu.sync_copy(enew_hbm.at[pl.ds(base + toff, TAIL)],
                        rows1.at[pl.ds(0, TAIL)])
        pltpu.sync_copy(rows1.at[pl.ds(0, TAIL)], acc.at[idxt], add=True)
        plsc.subcore_barrier()

        @pl.when((cid == 0) & (sid < NS - 1))
        def _():
            pltpu.sync_copy(acc.at[nsl], out0.at[nsl])

        @pl.when((cid == 0) & (sid == NS - 1))
        def _():
            pltpu.sync_copy(acc.at[nsl_last], out0.at[nsl_last])

        @pl.when((cid == 1) & (sid < NS - 1))
        def _():
            pltpu.sync_copy(acc.at[nsl], out1.at[nsl])

        @pl.when((cid == 1) & (sid == NS - 1))
        def _():
            pltpu.sync_copy(acc.at[nsl_last], out1.at[nsl_last])

    return scatter_add


def _gather_sum(hs, hr, senders, receivers):
    return _gather_sum_kernel()(hs, hr, senders, receivers)


def _scatter_add(e_new, receivers, zeros):
    return _scatter_add_kernel()(e_new, receivers, zeros)


def _r2(b):
    return b.reshape(1, -1)


def kernel(x, edge_index, edge_attr, velocity_sequence_noise, params):
    del velocity_sequence_noise
    frames = x[:, 1:3]
    node_type = x[:, 0].astype(jnp.int32)
    one_hot = jax.nn.one_hot(node_type, 9, dtype=jnp.float32)
    node_feats = jnp.concatenate([frames, one_hot], axis=1)
    nn = params["node_norm"]
    node_attr = (node_feats - nn["mean"]) / nn["std"]
    node_attr_p = jnp.pad(node_attr, ((0, 0), (0, 5)))
    edge_attr_p = jnp.pad(edge_attr, ((0, 0), (0, 4)))

    enb, eeb = params["enc_nb"], params["enc_eb"]
    h = _enc_call(node_attr_p,
                  jnp.pad(enb["l0"]["W"], ((0, 5), (0, 0))), _r2(enb["l0"]["b"]),
                  enb["l1"]["W"], _r2(enb["l1"]["b"]),
                  enb["l2"]["W"], _r2(enb["l2"]["b"]),
                  _r2(enb["ln"]["g"]), _r2(enb["ln"]["b"]), B_NODE)
    e = _enc_call(edge_attr_p,
                  jnp.pad(eeb["l0"]["W"], ((0, 4), (0, 0))), _r2(eeb["l0"]["b"]),
                  eeb["l1"]["W"], _r2(eeb["l1"]["b"]),
                  eeb["l2"]["W"], _r2(eeb["l2"]["b"]),
                  _r2(eeb["ln"]["g"]), _r2(eeb["ln"]["b"]), B_EDGE)

    senders = edge_index[0]
    receivers = edge_index[1]
    zeros = jnp.zeros((N_NODES, H), jnp.float32)

    for blk in params["blocks"]:
        eb, nb = blk["eb"], blk["nb"]
        w0 = eb["l0"]["W"]
        ws, wr, we = w0[:H], w0[H:2 * H], w0[2 * H:]
        hs, hr = _pre_call(h, ws, wr)
        gsum = _gather_sum(hs, hr, senders, receivers)
        e_new, e = _edge_call(gsum, e, we, _r2(eb["l0"]["b"]),
                              eb["l1"]["W"], _r2(eb["l1"]["b"]),
                              eb["l2"]["W"], _r2(eb["l2"]["b"]),
                              _r2(eb["ln"]["g"]), _r2(eb["ln"]["b"]))
        a0, a1 = _scatter_add(e_new, receivers, zeros)
        n0 = nb["l0"]["W"]
        h = _node_call(h, a0, a1, n0[:H], n0[H:], _r2(nb["l0"]["b"]),
                       nb["l1"]["W"], _r2(nb["l1"]["b"]),
                       nb["l2"]["W"], _r2(nb["l2"]["b"]),
                       _r2(nb["ln"]["g"]), _r2(nb["ln"]["b"]))

    dec = params["dec"]
    on = params["out_norm"]
    frames_p = jnp.pad(frames, ((0, 0), (0, 6)))
    w2p = jnp.pad(dec["l2"]["W"], ((0, 0), (0, 6)))
    b2p = jnp.pad(dec["l2"]["b"], (0, 6))
    stdp = jnp.pad(on["std"], (0, 6), constant_values=1.0)
    meanp = jnp.pad(on["mean"], (0, 6))
    out = _dec_call(h, frames_p,
                    dec["l0"]["W"], _r2(dec["l0"]["b"]),
                    dec["l1"]["W"], _r2(dec["l1"]["b"]),
                    w2p, _r2(b2p), _r2(stdp), _r2(meanp))
    return out[:, :2]


# speedup vs baseline: 4.4456x; 1.1297x over previous
import functools

import jax
import jax.numpy as jnp
from jax import lax
from jax.experimental import pallas as pl
from jax.experimental.pallas import tpu as pltpu
from jax.experimental.pallas import tpu_sc as plsc

N_NODES = 10000
N_EDGES = 160000
H = 128

B_NODE = 1000
B_EDGE = 4000

NC = 2
NS = 16
NW = NC * NS
BPW = N_EDGES // NW
CG = 128
NCH = BPW // CG
TAIL = BPW - NCH * CG
NPT = 632
NPT_LAST = N_NODES - (NS - 1) * NPT


def _ln(h, g, b):
    mu = jnp.mean(h, axis=-1, keepdims=True)
    var = jnp.mean((h - mu) * (h - mu), axis=-1, keepdims=True)
    return (h - mu) * lax.rsqrt(var + 1e-5) * g + b


def _dot(a, b):
    return jnp.dot(a, b, preferred_element_type=jnp.float32)


def _full(shape):
    return pl.BlockSpec(shape, lambda i: (0,) * len(shape))


def _enc_body(x_ref, w0, b0, w1, b1, w2, b2, g, bln, o_ref):
    h = jnp.maximum(_dot(x_ref[...], w0[...]) + b0[...], 0.0)
    h = jnp.maximum(_dot(h, w1[...]) + b1[...], 0.0)
    h = _dot(h, w2[...]) + b2[...]
    o_ref[...] = _ln(h, g[...], bln[...])


def _enc_pre_body(x_ref, w0, b0, w1, b1, w2, b2, g, bln, ws, wr,
                  o_ref, hs_ref, hr_ref):
    h = jnp.maximum(_dot(x_ref[...], w0[...]) + b0[...], 0.0)
    h = jnp.maximum(_dot(h, w1[...]) + b1[...], 0.0)
    h = _dot(h, w2[...]) + b2[...]
    out = _ln(h, g[...], bln[...])
    o_ref[...] = out
    hs_ref[...] = _dot(out, ws[...])
    hr_ref[...] = _dot(out, wr[...])


def _edge_body(g_ref, e_ref, we0, b0, w1, b1, w2, b2, g, bln,
               enew_ref, eout_ref):
    e = e_ref[...]
    h = jnp.maximum(g_ref[...] + _dot(e, we0[...]) + b0[...], 0.0)
    h = jnp.maximum(_dot(h, w1[...]) + b1[...], 0.0)
    h = _dot(h, w2[...]) + b2[...]
    enew = _ln(h, g[...], bln[...])
    enew_ref[...] = enew
    eout_ref[...] = e + enew


def _edge_last_body(g_ref, e_ref, we0, b0, w1, b1, w2, b2, g, bln,
                    enew_ref):
    e = e_ref[...]
    h = jnp.maximum(g_ref[...] + _dot(e, we0[...]) + b0[...], 0.0)
    h = jnp.maximum(_dot(h, w1[...]) + b1[...], 0.0)
    h = _dot(h, w2[...]) + b2[...]
    enew_ref[...] = _ln(h, g[...], bln[...])


def _node_body(h_ref, a0_ref, a1_ref, w0h, w0a, b0, w1, b1, w2, b2, g, bln,
               hout_ref):
    hin = h_ref[...]
    a = a0_ref[...] + a1_ref[...]
    h = jnp.maximum(_dot(hin, w0h[...]) + _dot(a, w0a[...]) + b0[...], 0.0)
    h = jnp.maximum(_dot(h, w1[...]) + b1[...], 0.0)
    h = _dot(h, w2[...]) + b2[...]
    hout_ref[...] = hin + _ln(h, g[...], bln[...])


def _node_pre_body(h_ref, a0_ref, a1_ref, w0h, w0a, b0, w1, b1, w2, b2,
                   g, bln, ws, wr, hout_ref, hs_ref, hr_ref):
    hin = h_ref[...]
    a = a0_ref[...] + a1_ref[...]
    h = jnp.maximum(_dot(hin, w0h[...]) + _dot(a, w0a[...]) + b0[...], 0.0)
    h = jnp.maximum(_dot(h, w1[...]) + b1[...], 0.0)
    h = _dot(h, w2[...]) + b2[...]
    hout = hin + _ln(h, g[...], bln[...])
    hout_ref[...] = hout
    hs_ref[...] = _dot(hout, ws[...])
    hr_ref[...] = _dot(hout, wr[...])


def _dec_body(h_ref, f_ref, w0, b0, w1, b1, w2, b2, std, mean, o_ref):
    h = jnp.maximum(_dot(h_ref[...], w0[...]) + b0[...], 0.0)
    h = jnp.maximum(_dot(h, w1[...]) + b1[...], 0.0)
    d = _dot(h, w2[...]) + b2[...]
    o_ref[...] = f_ref[...] + d * std[...] + mean[...]


def _row_spec(n_rows, b, k):
    return pl.BlockSpec((b, k), lambda i: (i, 0))


def _enc_call(x, w0, b0, w1, b1, w2, b2, g, bln, b_rows):
    n, k = x.shape
    grid = n // b_rows
    return pl.pallas_call(
        _enc_body,
        grid=(grid,),
        in_specs=[_row_spec(n, b_rows, k)] + [_full(w.shape) for w in
                  (w0, b0, w1, b1, w2, b2, g, bln)],
        out_specs=_row_spec(n, b_rows, H),
        out_shape=jax.ShapeDtypeStruct((n, H), jnp.float32),
    )(x, w0, b0, w1, b1, w2, b2, g, bln)


def _enc_pre_call(x, w0, b0, w1, b1, w2, b2, g, bln, ws, wr):
    n, k = x.shape
    grid = n // B_NODE
    spec = _row_spec(n, B_NODE, H)
    return pl.pallas_call(
        _enc_pre_body,
        grid=(grid,),
        in_specs=[_row_spec(n, B_NODE, k)] + [_full(w.shape) for w in
                  (w0, b0, w1, b1, w2, b2, g, bln, ws, wr)],
        out_specs=[spec, spec, spec],
        out_shape=[jax.ShapeDtypeStruct((n, H), jnp.float32)] * 3,
    )(x, w0, b0, w1, b1, w2, b2, g, bln, ws, wr)


def _edge_call(gsum, e, we0, b0, w1, b1, w2, b2, g, bln):
    grid = N_EDGES // B_EDGE
    spec = _row_spec(N_EDGES, B_EDGE, H)
    return pl.pallas_call(
        _edge_body,
        grid=(grid,),
        in_specs=[spec, spec] + [_full(w.shape) for w in
                  (we0, b0, w1, b1, w2, b2, g, bln)],
        out_specs=[spec, spec],
        out_shape=[jax.ShapeDtypeStruct((N_EDGES, H), jnp.float32)] * 2,
    )(gsum, e, we0, b0, w1, b1, w2, b2, g, bln)


def _edge_last_call(gsum, e, we0, b0, w1, b1, w2, b2, g, bln):
    grid = N_EDGES // B_EDGE
    spec = _row_spec(N_EDGES, B_EDGE, H)
    return pl.pallas_call(
        _edge_last_body,
        grid=(grid,),
        in_specs=[spec, spec] + [_full(w.shape) for w in
                  (we0, b0, w1, b1, w2, b2, g, bln)],
        out_specs=spec,
        out_shape=jax.ShapeDtypeStruct((N_EDGES, H), jnp.float32),
    )(gsum, e, we0, b0, w1, b1, w2, b2, g, bln)


def _node_call(h, a0, a1, w0h, w0a, b0, w1, b1, w2, b2, g, bln):
    grid = N_NODES // B_NODE
    spec = _row_spec(N_NODES, B_NODE, H)
    return pl.pallas_call(
        _node_body,
        grid=(grid,),
        in_specs=[spec, spec, spec] + [_full(w.shape) for w in
                  (w0h, w0a, b0, w1, b1, w2, b2, g, bln)],
        out_specs=spec,
        out_shape=jax.ShapeDtypeStruct((N_NODES, H), jnp.float32),
    )(h, a0, a1, w0h, w0a, b0, w1, b1, w2, b2, g, bln)


def _node_pre_call(h, a0, a1, w0h, w0a, b0, w1, b1, w2, b2, g, bln, ws, wr):
    grid = N_NODES // B_NODE
    spec = _row_spec(N_NODES, B_NODE, H)
    return pl.pallas_call(
        _node_pre_body,
        grid=(grid,),
        in_specs=[spec, spec, spec] + [_full(w.shape) for w in
                  (w0h, w0a, b0, w1, b1, w2, b2, g, bln, ws, wr)],
        out_specs=[spec, spec, spec],
        out_shape=[jax.ShapeDtypeStruct((N_NODES, H), jnp.float32)] * 3,
    )(h, a0, a1, w0h, w0a, b0, w1, b1, w2, b2, g, bln, ws, wr)


def _dec_call(h, frames_p, w0, b0, w1, b1, w2, b2, std, mean):
    grid = N_NODES // B_NODE
    return pl.pallas_call(
        _dec_body,
        grid=(grid,),
        in_specs=[_row_spec(N_NODES, B_NODE, H),
                  _row_spec(N_NODES, B_NODE, 8)] +
                 [_full(w.shape) for w in (w0, b0, w1, b1, w2, b2, std, mean)],
        out_specs=_row_spec(N_NODES, B_NODE, 8),
        out_shape=jax.ShapeDtypeStruct((N_NODES, 8), jnp.float32),
    )(h, frames_p, w0, b0, w1, b1, w2, b2, std, mean)


def _sc_mesh():
    return plsc.VectorSubcoreMesh(core_axis_name="c", subcore_axis_name="s",
                                  num_cores=NC, num_subcores=NS)


@functools.cache
def _gather_sum_kernel():
    @functools.partial(
        pl.kernel,
        out_type=jax.ShapeDtypeStruct((N_EDGES, H), jnp.float32),
        mesh=_sc_mesh(),
        scratch_types=[
            pltpu.VMEM((BPW,), jnp.int32),
            pltpu.VMEM((BPW,), jnp.int32),
            pltpu.VMEM((CG, H), jnp.float32),
            pltpu.VMEM((CG, H), jnp.float32),
            pltpu.VMEM((CG, H), jnp.float32),
            pltpu.VMEM((CG, H), jnp.float32),
            pltpu.SemaphoreType.DMA,
            pltpu.SemaphoreType.DMA,
            pltpu.SemaphoreType.DMA,
            pltpu.SemaphoreType.DMA,
        ],
    )
    def gather_sum(hs_hbm, hr_hbm, s_hbm, r_hbm, out_hbm,
                   sidx, ridx, srows0, rrows0, srows1, rrows1,
                   ss0, sr0, ss1, sr1):
        cid = lax.axis_index("c")
        sid = lax.axis_index("s")
        base = (sid * NC + cid) * BPW
        pltpu.sync_copy(s_hbm.at[pl.ds(base, BPW)], sidx)
        pltpu.sync_copy(r_hbm.at[pl.ds(base, BPW)], ridx)
        bufs = ((srows0, rrows0, ss0, sr0), (srows1, rrows1, ss1, sr1))

        def issue(off, b):
            sb, rb, ss, sr = bufs[b]
            pltpu.async_copy(hs_hbm.at[sidx.at[pl.ds(off, CG)]], sb, ss)
            pltpu.async_copy(hr_hbm.at[ridx.at[pl.ds(off, CG)]], rb, sr)

        def drain_compute(off, b):
            sb, rb, ss, sr = bufs[b]
            pltpu.make_async_copy(hs_hbm.at[sidx.at[pl.ds(off, CG)]], sb,
                                  ss).wait()
            pltpu.make_async_copy(hr_hbm.at[ridx.at[pl.ds(off, CG)]], rb,
                                  sr).wait()

            def row(i, c):
                for j in range(H // 16):
                    sl = pl.ds(j * 16, 16)
                    sb[i, sl] = sb[i, sl] + rb[i, sl]
                return c
            lax.fori_loop(0, CG, row, 0)
            pltpu.sync_copy(sb, out_hbm.at[pl.ds(base + off, CG)])

        issue(0, 0)
        issue(CG, 1)

        def round_(g, c):
            k = g * 2
            drain_compute(k * CG, 0)

            @pl.when(k + 2 < NCH)
            def _():
                issue((k + 2) * CG, 0)
            drain_compute((k + 1) * CG, 1)

            @pl.when(k + 3 < NCH)
            def _():
                issue((k + 3) * CG, 1)
            return c
        lax.fori_loop(0, (NCH - 1) // 2, round_, 0)
        drain_compute((NCH - 1) * CG, 0)

        sb, rb, ss, _ = bufs[1]
        toff = NCH * CG
        pltpu.async_copy(hs_hbm.at[sidx.at[pl.ds(toff, TAIL)]],
                         sb.at[pl.ds(0, TAIL)], ss).wait()
        pltpu.async_copy(hr_hbm.at[ridx.at[pl.ds(toff, TAIL)]],
                         rb.at[pl.ds(0, TAIL)], ss).wait()

        def trow(i, c):
            for j in range(H // 16):
                sl = pl.ds(j * 16, 16)
                sb[i, sl] = sb[i, sl] + rb[i, sl]
            return c
        lax.fori_loop(0, TAIL, trow, 0)
        pltpu.sync_copy(sb.at[pl.ds(0, TAIL)],
                        out_hbm.at[pl.ds(base + toff, TAIL)])

    return gather_sum


@functools.cache
def _scatter_add_kernel():
    @functools.partial(
        pl.kernel,
        out_type=(jax.ShapeDtypeStruct((N_NODES, H), jnp.float32),
                  jax.ShapeDtypeStruct((N_NODES, H), jnp.float32)),
        mesh=_sc_mesh(),
        scratch_types=[
            pltpu.VMEM_SHARED((N_NODES, H), jnp.float32),
            pltpu.VMEM((CG,), jnp.int32),
            pltpu.VMEM((CG,), jnp.int32),
            pltpu.VMEM((TAIL,), jnp.int32),
            pltpu.VMEM((CG, H), jnp.float32),
            pltpu.VMEM((CG, H), jnp.float32),
            pltpu.SemaphoreType.DMA,
            pltpu.SemaphoreType.DMA,
            pltpu.SemaphoreType.DMA,
            pltpu.SemaphoreType.DMA,
        ],
    )
    def scatter_add(enew_hbm, r_hbm, zeros_hbm, out0, out1,
                    acc, idxc0, idxc1, idxt, rows0, rows1,
                    si0, sd0, si1, sd1):
        cid = lax.axis_index("c")
        sid = lax.axis_index("s")
        nsl = pl.ds(sid * NPT, NPT)
        nsl_last = pl.ds((NS - 1) * NPT, NPT_LAST)

        @pl.when(sid < NS - 1)
        def _():
            pltpu.sync_copy(zeros_hbm.at[nsl], acc.at[nsl])

        @pl.when(sid == NS - 1)
        def _():
            pltpu.sync_copy(zeros_hbm.at[nsl_last], acc.at[nsl_last])
        plsc.subcore_barrier()

        base = cid * (N_EDGES // NC) + sid * BPW
        bufs = ((idxc0, rows0, si0, sd0), (idxc1, rows1, si1, sd1))

        def issue(off, b):
            ib, rb, si, sd = bufs[b]
            pltpu.async_copy(r_hbm.at[pl.ds(base + off, CG)], ib, si)
            pltpu.async_copy(enew_hbm.at[pl.ds(base + off, CG)], rb, sd)

        def drain_scatter(off, b):
            ib, rb, si, sd = bufs[b]
            pltpu.make_async_copy(r_hbm.at[pl.ds(base + off, CG)], ib,
                                  si).wait()
            pltpu.make_async_copy(enew_hbm.at[pl.ds(base + off, CG)], rb,
                                  sd).wait()
            pltpu.sync_copy(rb, acc.at[ib], add=True)

        issue(0, 0)
        issue(CG, 1)

        def round_(g, c):
            k = g * 2
            drain_scatter(k * CG, 0)

            @pl.when(k + 2 < NCH)
            def _():
                issue((k + 2) * CG, 0)
            drain_scatter((k + 1) * CG, 1)

            @pl.when(k + 3 < NCH)
            def _():
                issue((k + 3) * CG, 1)
            return c
        lax.fori_loop(0, (NCH - 1) // 2, round_, 0)
        drain_scatter((NCH - 1) * CG, 0)

        toff = NCH * CG
        pltpu.sync_copy(r_hbm.at[pl.ds(base + toff, TAIL)], idxt)
        pltpu.sync_copy(enew_hbm.at[pl.ds(base + toff, TAIL)],
                        rows1.at[pl.ds(0, TAIL)])
        pltpu.sync_copy(rows1.at[pl.ds(0, TAIL)], acc.at[idxt], add=True)
        plsc.subcore_barrier()

        @pl.when((cid == 0) & (sid < NS - 1))
        def _():
            pltpu.sync_copy(acc.at[nsl], out0.at[nsl])

        @pl.when((cid == 0) & (sid == NS - 1))
        def _():
            pltpu.sync_copy(acc.at[nsl_last], out0.at[nsl_last])

        @pl.when((cid == 1) & (sid < NS - 1))
        def _():
            pltpu.sync_copy(acc.at[nsl], out1.at[nsl])

        @pl.when((cid == 1) & (sid == NS - 1))
        def _():
            pltpu.sync_copy(acc.at[nsl_last], out1.at[nsl_last])

    return scatter_add


def _gather_sum(hs, hr, senders, receivers):
    return _gather_sum_kernel()(hs, hr, senders, receivers)


def _scatter_add(e_new, receivers, zeros):
    return _scatter_add_kernel()(e_new, receivers, zeros)


def _r2(b):
    return b.reshape(1, -1)


def kernel(x, edge_index, edge_attr, velocity_sequence_noise, params):
    del velocity_sequence_noise
    frames = x[:, 1:3]
    node_type = x[:, 0].astype(jnp.int32)
    one_hot = jax.nn.one_hot(node_type, 9, dtype=jnp.float32)
    node_feats = jnp.concatenate([frames, one_hot], axis=1)
    nn = params["node_norm"]
    node_attr = (node_feats - nn["mean"]) / nn["std"]
    node_attr_p = jnp.pad(node_attr, ((0, 0), (0, 5)))
    edge_attr_p = jnp.pad(edge_attr, ((0, 0), (0, 4)))

    blocks = params["blocks"]
    splits = [blk["eb"]["l0"]["W"] for blk in blocks]
    enb, eeb = params["enc_nb"], params["enc_eb"]
    h, hs, hr = _enc_pre_call(
        node_attr_p,
        jnp.pad(enb["l0"]["W"], ((0, 5), (0, 0))), _r2(enb["l0"]["b"]),
        enb["l1"]["W"], _r2(enb["l1"]["b"]),
        enb["l2"]["W"], _r2(enb["l2"]["b"]),
        _r2(enb["ln"]["g"]), _r2(enb["ln"]["b"]),
        splits[0][:H], splits[0][H:2 * H])
    e = _enc_call(edge_attr_p,
                  jnp.pad(eeb["l0"]["W"], ((0, 4), (0, 0))), _r2(eeb["l0"]["b"]),
                  eeb["l1"]["W"], _r2(eeb["l1"]["b"]),
                  eeb["l2"]["W"], _r2(eeb["l2"]["b"]),
                  _r2(eeb["ln"]["g"]), _r2(eeb["ln"]["b"]), B_EDGE)

    senders = edge_index[0]
    receivers = edge_index[1]
    zeros = jnp.zeros((N_NODES, H), jnp.float32)

    for k, blk in enumerate(blocks):
        eb, nb = blk["eb"], blk["nb"]
        last = k == len(blocks) - 1
        we = splits[k][2 * H:]
        gsum = _gather_sum(hs, hr, senders, receivers)
        eargs = (gsum, e, we, _r2(eb["l0"]["b"]),
                 eb["l1"]["W"], _r2(eb["l1"]["b"]),
                 eb["l2"]["W"], _r2(eb["l2"]["b"]),
                 _r2(eb["ln"]["g"]), _r2(eb["ln"]["b"]))
        if last:
            e_new = _edge_last_call(*eargs)
        else:
            e_new, e = _edge_call(*eargs)
        a0, a1 = _scatter_add(e_new, receivers, zeros)
        n0 = nb["l0"]["W"]
        nargs = (h, a0, a1, n0[:H], n0[H:], _r2(nb["l0"]["b"]),
                 nb["l1"]["W"], _r2(nb["l1"]["b"]),
                 nb["l2"]["W"], _r2(nb["l2"]["b"]),
                 _r2(nb["ln"]["g"]), _r2(nb["ln"]["b"]))
        if last:
            h = _node_call(*nargs)
        else:
            h, hs, hr = _node_pre_call(*nargs, splits[k + 1][:H],
                                       splits[k + 1][H:2 * H])

    dec = params["dec"]
    on = params["out_norm"]
    frames_p = jnp.pad(frames, ((0, 0), (0, 6)))
    w2p = jnp.pad(dec["l2"]["W"], ((0, 0), (0, 6)))
    b2p = jnp.pad(dec["l2"]["b"], (0, 6))
    stdp = jnp.pad(on["std"], (0, 6), constant_values=1.0)
    meanp = jnp.pad(on["mean"], (0, 6))
    out = _dec_call(h, frames_p,
                    dec["l0"]["W"], _r2(dec["l0"]["b"]),
                    dec["l1"]["W"], _r2(dec["l1"]["b"]),
                    w2p, _r2(b2p), _r2(stdp), _r2(meanp))
    return out[:, :2]


# speedup vs baseline: 4.5155x; 1.0157x over previous
import functools

import jax
import jax.numpy as jnp
from jax import lax
from jax.experimental import pallas as pl
from jax.experimental.pallas import tpu as pltpu
from jax.experimental.pallas import tpu_sc as plsc

N_NODES = 10000
N_EDGES = 160000
H = 128

B_NODE = 1000
B_EDGE = 4000

NC = 2
NS = 16
NW = NC * NS
BPW = N_EDGES // NW
CG = 128
NCH = BPW // CG
TAIL = BPW - NCH * CG
NPT = 632
NPT_LAST = N_NODES - (NS - 1) * NPT


def _ln(h, g, b):
    mu = jnp.mean(h, axis=-1, keepdims=True)
    var = jnp.mean((h - mu) * (h - mu), axis=-1, keepdims=True)
    return (h - mu) * lax.rsqrt(var + 1e-5) * g + b


def _dot(a, b):
    return jnp.dot(a, b, preferred_element_type=jnp.float32)


def _full(shape):
    return pl.BlockSpec(shape, lambda i: (0,) * len(shape))


def _enc_body(x_ref, w0, b0, w1, b1, w2, b2, g, bln, o_ref):
    h = jnp.maximum(_dot(x_ref[...], w0[...]) + b0[...], 0.0)
    h = jnp.maximum(_dot(h, w1[...]) + b1[...], 0.0)
    h = _dot(h, w2[...]) + b2[...]
    o_ref[...] = _ln(h, g[...], bln[...])


def _enc_pre_body(x_ref, w0, b0, w1, b1, w2, b2, g, bln, ws, wr,
                  o_ref, hs_ref, hr_ref):
    h = jnp.maximum(_dot(x_ref[...], w0[...]) + b0[...], 0.0)
    h = jnp.maximum(_dot(h, w1[...]) + b1[...], 0.0)
    h = _dot(h, w2[...]) + b2[...]
    out = _ln(h, g[...], bln[...])
    o_ref[...] = out
    hs_ref[...] = _dot(out, ws[...])
    hr_ref[...] = _dot(out, wr[...])


def _edge_body(g_ref, e_ref, we0, b0, w1, b1, w2, b2, g, bln,
               enew_ref, eout_ref):
    e = e_ref[...]
    h = jnp.maximum(g_ref[...] + _dot(e, we0[...]) + b0[...], 0.0)
    h = jnp.maximum(_dot(h, w1[...]) + b1[...], 0.0)
    h = _dot(h, w2[...]) + b2[...]
    enew = _ln(h, g[...], bln[...])
    enew_ref[...] = enew
    eout_ref[...] = e + enew


def _edge_last_body(g_ref, e_ref, we0, b0, w1, b1, w2, b2, g, bln,
                    enew_ref):
    e = e_ref[...]
    h = jnp.maximum(g_ref[...] + _dot(e, we0[...]) + b0[...], 0.0)
    h = jnp.maximum(_dot(h, w1[...]) + b1[...], 0.0)
    h = _dot(h, w2[...]) + b2[...]
    enew_ref[...] = _ln(h, g[...], bln[...])


def _node_body(h_ref, a0_ref, a1_ref, w0h, w0a, b0, w1, b1, w2, b2, g, bln,
               hout_ref):
    hin = h_ref[...]
    a = a0_ref[...] + a1_ref[...]
    h = jnp.maximum(_dot(hin, w0h[...]) + _dot(a, w0a[...]) + b0[...], 0.0)
    h = jnp.maximum(_dot(h, w1[...]) + b1[...], 0.0)
    h = _dot(h, w2[...]) + b2[...]
    hout_ref[...] = hin + _ln(h, g[...], bln[...])


def _node_pre_body(h_ref, a0_ref, a1_ref, w0h, w0a, b0, w1, b1, w2, b2,
                   g, bln, ws, wr, hout_ref, hs_ref, hr_ref):
    hin = h_ref[...]
    a = a0_ref[...] + a1_ref[...]
    h = jnp.maximum(_dot(hin, w0h[...]) + _dot(a, w0a[...]) + b0[...], 0.0)
    h = jnp.maximum(_dot(h, w1[...]) + b1[...], 0.0)
    h = _dot(h, w2[...]) + b2[...]
    hout = hin + _ln(h, g[...], bln[...])
    hout_ref[...] = hout
    hs_ref[...] = _dot(hout, ws[...])
    hr_ref[...] = _dot(hout, wr[...])


def _dec_body(h_ref, f_ref, w0, b0, w1, b1, w2, b2, std, mean, o_ref):
    h = jnp.maximum(_dot(h_ref[...], w0[...]) + b0[...], 0.0)
    h = jnp.maximum(_dot(h, w1[...]) + b1[...], 0.0)
    d = _dot(h, w2[...]) + b2[...]
    o_ref[...] = f_ref[...] + d * std[...] + mean[...]


def _row_spec(n_rows, b, k):
    return pl.BlockSpec((b, k), lambda i: (i, 0))


def _enc_call(x, w0, b0, w1, b1, w2, b2, g, bln, b_rows):
    n, k = x.shape
    grid = n // b_rows
    return pl.pallas_call(
        _enc_body,
        grid=(grid,),
        in_specs=[_row_spec(n, b_rows, k)] + [_full(w.shape) for w in
                  (w0, b0, w1, b1, w2, b2, g, bln)],
        out_specs=_row_spec(n, b_rows, H),
        out_shape=jax.ShapeDtypeStruct((n, H), jnp.float32),
    )(x, w0, b0, w1, b1, w2, b2, g, bln)


def _enc_pre_call(x, w0, b0, w1, b1, w2, b2, g, bln, ws, wr):
    n, k = x.shape
    grid = n // B_NODE
    spec = _row_spec(n, B_NODE, H)
    return pl.pallas_call(
        _enc_pre_body,
        grid=(grid,),
        in_specs=[_row_spec(n, B_NODE, k)] + [_full(w.shape) for w in
                  (w0, b0, w1, b1, w2, b2, g, bln, ws, wr)],
        out_specs=[spec, spec, spec],
        out_shape=[jax.ShapeDtypeStruct((n, H), jnp.float32)] * 3,
    )(x, w0, b0, w1, b1, w2, b2, g, bln, ws, wr)


def _edge_call(gsum, e, we0, b0, w1, b1, w2, b2, g, bln):
    grid = N_EDGES // B_EDGE
    spec = _row_spec(N_EDGES, B_EDGE, H)
    return pl.pallas_call(
        _edge_body,
        grid=(grid,),
        in_specs=[spec, spec] + [_full(w.shape) for w in
                  (we0, b0, w1, b1, w2, b2, g, bln)],
        out_specs=[spec, spec],
        out_shape=[jax.ShapeDtypeStruct((N_EDGES, H), jnp.float32)] * 2,
    )(gsum, e, we0, b0, w1, b1, w2, b2, g, bln)


def _edge_last_call(gsum, e, we0, b0, w1, b1, w2, b2, g, bln):
    grid = N_EDGES // B_EDGE
    spec = _row_spec(N_EDGES, B_EDGE, H)
    return pl.pallas_call(
        _edge_last_body,
        grid=(grid,),
        in_specs=[spec, spec] + [_full(w.shape) for w in
                  (we0, b0, w1, b1, w2, b2, g, bln)],
        out_specs=spec,
        out_shape=jax.ShapeDtypeStruct((N_EDGES, H), jnp.float32),
    )(gsum, e, we0, b0, w1, b1, w2, b2, g, bln)


def _node_call(h, a0, a1, w0h, w0a, b0, w1, b1, w2, b2, g, bln):
    grid = N_NODES // B_NODE
    spec = _row_spec(N_NODES, B_NODE, H)
    return pl.pallas_call(
        _node_body,
        grid=(grid,),
        in_specs=[spec, spec, spec] + [_full(w.shape) for w in
                  (w0h, w0a, b0, w1, b1, w2, b2, g, bln)],
        out_specs=spec,
        out_shape=jax.ShapeDtypeStruct((N_NODES, H), jnp.float32),
    )(h, a0, a1, w0h, w0a, b0, w1, b1, w2, b2, g, bln)


def _node_pre_call(h, a0, a1, w0h, w0a, b0, w1, b1, w2, b2, g, bln, ws, wr):
    grid = N_NODES // B_NODE
    spec = _row_spec(N_NODES, B_NODE, H)
    return pl.pallas_call(
        _node_pre_body,
        grid=(grid,),
        in_specs=[spec, spec, spec] + [_full(w.shape) for w in
                  (w0h, w0a, b0, w1, b1, w2, b2, g, bln, ws, wr)],
        out_specs=[spec, spec, spec],
        out_shape=[jax.ShapeDtypeStruct((N_NODES, H), jnp.float32)] * 3,
    )(h, a0, a1, w0h, w0a, b0, w1, b1, w2, b2, g, bln, ws, wr)


def _dec_call(h, frames_p, w0, b0, w1, b1, w2, b2, std, mean):
    grid = N_NODES // B_NODE
    return pl.pallas_call(
        _dec_body,
        grid=(grid,),
        in_specs=[_row_spec(N_NODES, B_NODE, H),
                  _row_spec(N_NODES, B_NODE, 8)] +
                 [_full(w.shape) for w in (w0, b0, w1, b1, w2, b2, std, mean)],
        out_specs=_row_spec(N_NODES, B_NODE, 8),
        out_shape=jax.ShapeDtypeStruct((N_NODES, 8), jnp.float32),
    )(h, frames_p, w0, b0, w1, b1, w2, b2, std, mean)


def _sc_mesh():
    return plsc.VectorSubcoreMesh(core_axis_name="c", subcore_axis_name="s",
                                  num_cores=NC, num_subcores=NS)


@functools.cache
def _gather_sum_kernel():
    @functools.partial(
        pl.kernel,
        out_type=jax.ShapeDtypeStruct((N_EDGES, H), jnp.float32),
        mesh=_sc_mesh(),
        scratch_types=[
            pltpu.VMEM((BPW,), jnp.int32),
            pltpu.VMEM((BPW,), jnp.int32),
            pltpu.VMEM((CG, H), jnp.float32),
            pltpu.VMEM((CG, H), jnp.float32),
            pltpu.VMEM((CG, H), jnp.float32),
            pltpu.VMEM((CG, H), jnp.float32),
            pltpu.VMEM((CG, H), jnp.float32),
            pltpu.VMEM((CG, H), jnp.float32),
            pltpu.SemaphoreType.DMA,
            pltpu.SemaphoreType.DMA,
            pltpu.SemaphoreType.DMA,
            pltpu.SemaphoreType.DMA,
            pltpu.SemaphoreType.DMA,
            pltpu.SemaphoreType.DMA,
        ],
    )
    def gather_sum(hs_hbm, hr_hbm, s_hbm, r_hbm, out_hbm,
                   sidx, ridx, srows0, rrows0, srows1, rrows1,
                   srows2, rrows2, ss0, sr0, ss1, sr1, ss2, sr2):
        cid = lax.axis_index("c")
        sid = lax.axis_index("s")
        base = (sid * NC + cid) * BPW
        pltpu.sync_copy(s_hbm.at[pl.ds(base, BPW)], sidx)
        pltpu.sync_copy(r_hbm.at[pl.ds(base, BPW)], ridx)
        bufs = ((srows0, rrows0, ss0, sr0), (srows1, rrows1, ss1, sr1),
                (srows2, rrows2, ss2, sr2))

        def issue(off, b):
            sb, rb, ss, sr = bufs[b]
            pltpu.async_copy(hs_hbm.at[sidx.at[pl.ds(off, CG)]], sb, ss)
            pltpu.async_copy(hr_hbm.at[ridx.at[pl.ds(off, CG)]], rb, sr)

        def drain_compute(off, b):
            sb, rb, ss, sr = bufs[b]
            pltpu.make_async_copy(hs_hbm.at[sidx.at[pl.ds(off, CG)]], sb,
                                  ss).wait()
            pltpu.make_async_copy(hr_hbm.at[ridx.at[pl.ds(off, CG)]], rb,
                                  sr).wait()

            def row(i, c):
                for j in range(H // 16):
                    sl = pl.ds(j * 16, 16)
                    sb[i, sl] = sb[i, sl] + rb[i, sl]
                return c
            lax.fori_loop(0, CG, row, 0)
            pltpu.sync_copy(sb, out_hbm.at[pl.ds(base + off, CG)])

        for b in range(3):
            issue(b * CG, b)

        def round_(g, c):
            for b in range(3):
                k = g * 3 + b
                drain_compute(k * CG, b)
                issue((k + 3) * CG, b)
            return c
        lax.fori_loop(0, NCH // 3 - 1, round_, 0)
        for b in range(3):
            drain_compute((NCH - 3 + b) * CG, b)

        sb, rb, ss, _ = bufs[1]
        toff = NCH * CG
        pltpu.async_copy(hs_hbm.at[sidx.at[pl.ds(toff, TAIL)]],
                         sb.at[pl.ds(0, TAIL)], ss).wait()
        pltpu.async_copy(hr_hbm.at[ridx.at[pl.ds(toff, TAIL)]],
                         rb.at[pl.ds(0, TAIL)], ss).wait()

        def trow(i, c):
            for j in range(H // 16):
                sl = pl.ds(j * 16, 16)
                sb[i, sl] = sb[i, sl] + rb[i, sl]
            return c
        lax.fori_loop(0, TAIL, trow, 0)
        pltpu.sync_copy(sb.at[pl.ds(0, TAIL)],
                        out_hbm.at[pl.ds(base + toff, TAIL)])

    return gather_sum


@functools.cache
def _scatter_add_kernel():
    @functools.partial(
        pl.kernel,
        out_type=(jax.ShapeDtypeStruct((N_NODES, H), jnp.float32),
                  jax.ShapeDtypeStruct((N_NODES, H), jnp.float32)),
        mesh=_sc_mesh(),
        scratch_types=[
            pltpu.VMEM_SHARED((N_NODES, H), jnp.float32),
            pltpu.VMEM((CG,), jnp.int32),
            pltpu.VMEM((CG,), jnp.int32),
            pltpu.VMEM((CG,), jnp.int32),
            pltpu.VMEM((TAIL,), jnp.int32),
            pltpu.VMEM((CG, H), jnp.float32),
            pltpu.VMEM((CG, H), jnp.float32),
            pltpu.VMEM((CG, H), jnp.float32),
            pltpu.SemaphoreType.DMA,
            pltpu.SemaphoreType.DMA,
            pltpu.SemaphoreType.DMA,
            pltpu.SemaphoreType.DMA,
            pltpu.SemaphoreType.DMA,
            pltpu.SemaphoreType.DMA,
        ],
    )
    def scatter_add(enew_hbm, r_hbm, zeros_hbm, out0, out1,
                    acc, idxc0, idxc1, idxc2, idxt, rows0, rows1, rows2,
                    si0, sd0, si1, sd1, si2, sd2):
        cid = lax.axis_index("c")
        sid = lax.axis_index("s")
        nsl = pl.ds(sid * NPT, NPT)
        nsl_last = pl.ds((NS - 1) * NPT, NPT_LAST)

        @pl.when(sid < NS - 1)
        def _():
            pltpu.sync_copy(zeros_hbm.at[nsl], acc.at[nsl])

        @pl.when(sid == NS - 1)
        def _():
            pltpu.sync_copy(zeros_hbm.at[nsl_last], acc.at[nsl_last])
        plsc.subcore_barrier()

        base = cid * (N_EDGES // NC) + sid * BPW
        bufs = ((idxc0, rows0, si0, sd0), (idxc1, rows1, si1, sd1),
                (idxc2, rows2, si2, sd2))

        def issue(off, b):
            ib, rb, si, sd = bufs[b]
            pltpu.async_copy(r_hbm.at[pl.ds(base + off, CG)], ib, si)
            pltpu.async_copy(enew_hbm.at[pl.ds(base + off, CG)], rb, sd)

        def drain_scatter(off, b):
            ib, rb, si, sd = bufs[b]
            pltpu.make_async_copy(r_hbm.at[pl.ds(base + off, CG)], ib,
                                  si).wait()
            pltpu.make_async_copy(enew_hbm.at[pl.ds(base + off, CG)], rb,
                                  sd).wait()
            pltpu.sync_copy(rb, acc.at[ib], add=True)

        for b in range(3):
            issue(b * CG, b)

        def round_(g, c):
            for b in range(3):
                k = g * 3 + b
                drain_scatter(k * CG, b)
                issue((k + 3) * CG, b)
            return c
        lax.fori_loop(0, NCH // 3 - 1, round_, 0)
        for b in range(3):
            drain_scatter((NCH - 3 + b) * CG, b)

        toff = NCH * CG
        pltpu.sync_copy(r_hbm.at[pl.ds(base + toff, TAIL)], idxt)
        pltpu.sync_copy(enew_hbm.at[pl.ds(base + toff, TAIL)],
                        rows1.at[pl.ds(0, TAIL)])
        pltpu.sync_copy(rows1.at[pl.ds(0, TAIL)], acc.at[idxt], add=True)
        plsc.subcore_barrier()

        @pl.when((cid == 0) & (sid < NS - 1))
        def _():
            pltpu.sync_copy(acc.at[nsl], out0.at[nsl])

        @pl.when((cid == 0) & (sid == NS - 1))
        def _():
            pltpu.sync_copy(acc.at[nsl_last], out0.at[nsl_last])

        @pl.when((cid == 1) & (sid < NS - 1))
        def _():
            pltpu.sync_copy(acc.at[nsl], out1.at[nsl])

        @pl.when((cid == 1) & (sid == NS - 1))
        def _():
            pltpu.sync_copy(acc.at[nsl_last], out1.at[nsl_last])

    return scatter_add


def _gather_sum(hs, hr, senders, receivers):
    return _gather_sum_kernel()(hs, hr, senders, receivers)


def _scatter_add(e_new, receivers, zeros):
    return _scatter_add_kernel()(e_new, receivers, zeros)


def _r2(b):
    return b.reshape(1, -1)


def kernel(x, edge_index, edge_attr, velocity_sequence_noise, params):
    del velocity_sequence_noise
    frames = x[:, 1:3]
    node_type = x[:, 0].astype(jnp.int32)
    one_hot = jax.nn.one_hot(node_type, 9, dtype=jnp.float32)
    node_feats = jnp.concatenate([frames, one_hot], axis=1)
    nn = params["node_norm"]
    node_attr = (node_feats - nn["mean"]) / nn["std"]
    node_attr_p = jnp.pad(node_attr, ((0, 0), (0, 5)))
    edge_attr_p = jnp.pad(edge_attr, ((0, 0), (0, 4)))

    blocks = params["blocks"]
    splits = [blk["eb"]["l0"]["W"] for blk in blocks]
    enb, eeb = params["enc_nb"], params["enc_eb"]
    h, hs, hr = _enc_pre_call(
        node_attr_p,
        jnp.pad(enb["l0"]["W"], ((0, 5), (0, 0))), _r2(enb["l0"]["b"]),
        enb["l1"]["W"], _r2(enb["l1"]["b"]),
        enb["l2"]["W"], _r2(enb["l2"]["b"]),
        _r2(enb["ln"]["g"]), _r2(enb["ln"]["b"]),
        splits[0][:H], splits[0][H:2 * H])
    e = _enc_call(edge_attr_p,
                  jnp.pad(eeb["l0"]["W"], ((0, 4), (0, 0))), _r2(eeb["l0"]["b"]),
                  eeb["l1"]["W"], _r2(eeb["l1"]["b"]),
                  eeb["l2"]["W"], _r2(eeb["l2"]["b"]),
                  _r2(eeb["ln"]["g"]), _r2(eeb["ln"]["b"]), B_EDGE)

    senders = edge_index[0]
    receivers = edge_index[1]
    zeros = jnp.zeros((N_NODES, H), jnp.float32)

    for k, blk in enumerate(blocks):
        eb, nb = blk["eb"], blk["nb"]
        last = k == len(blocks) - 1
        we = splits[k][2 * H:]
        gsum = _gather_sum(hs, hr, senders, receivers)
        eargs = (gsum, e, we, _r2(eb["l0"]["b"]),
                 eb["l1"]["W"], _r2(eb["l1"]["b"]),
                 eb["l2"]["W"], _r2(eb["l2"]["b"]),
                 _r2(eb["ln"]["g"]), _r2(eb["ln"]["b"]))
        if last:
            e_new = _edge_last_call(*eargs)
        else:
            e_new, e = _edge_call(*eargs)
        a0, a1 = _scatter_add(e_new, receivers, zeros)
        n0 = nb["l0"]["W"]
        nargs = (h, a0, a1, n0[:H], n0[H:], _r2(nb["l0"]["b"]),
                 nb["l1"]["W"], _r2(nb["l1"]["b"]),
                 nb["l2"]["W"], _r2(nb["l2"]["b"]),
                 _r2(nb["ln"]["g"]), _r2(nb["ln"]["b"]))
        if last:
            h = _node_call(*nargs)
        else:
            h, hs, hr = _node_pre_call(*nargs, splits[k + 1][:H],
                                       splits[k + 1][H:2 * H])

    dec = params["dec"]
    on = params["out_norm"]
    frames_p = jnp.pad(frames, ((0, 0), (0, 6)))
    w2p = jnp.pad(dec["l2"]["W"], ((0, 0), (0, 6)))
    b2p = jnp.pad(dec["l2"]["b"], (0, 6))
    stdp = jnp.pad(on["std"], (0, 6), constant_values=1.0)
    meanp = jnp.pad(on["mean"], (0, 6))
    out = _dec_call(h, frames_p,
                    dec["l0"]["W"], _r2(dec["l0"]["b"]),
                    dec["l1"]["W"], _r2(dec["l1"]["b"]),
                    w2p, _r2(b2p), _r2(stdp), _r2(meanp))
    return out[:, :2]


# speedup vs baseline: 4.7151x; 1.0442x over previous
import functools

import jax
import jax.numpy as jnp
from jax import lax
from jax.experimental import pallas as pl
from jax.experimental.pallas import tpu as pltpu
from jax.experimental.pallas import tpu_sc as plsc

N_NODES = 10000
N_EDGES = 160000
H = 128

B_NODE = 2000
B_EDGE = 8000

NC = 2
NS = 16
NW = NC * NS
BPW = N_EDGES // NW
CG = 128
NCH = BPW // CG
TAIL = BPW - NCH * CG
NPT = 632
NPT_LAST = N_NODES - (NS - 1) * NPT


def _ln(h, g, b):
    mu = jnp.mean(h, axis=-1, keepdims=True)
    var = jnp.mean((h - mu) * (h - mu), axis=-1, keepdims=True)
    return (h - mu) * lax.rsqrt(var + 1e-5) * g + b


def _dot(a, b):
    return jnp.dot(a, b, preferred_element_type=jnp.float32)


def _full(shape):
    return pl.BlockSpec(shape, lambda i: (0,) * len(shape))


def _enc_body(x_ref, w0, b0, w1, b1, w2, b2, g, bln, o_ref):
    h = jnp.maximum(_dot(x_ref[...], w0[...]) + b0[...], 0.0)
    h = jnp.maximum(_dot(h, w1[...]) + b1[...], 0.0)
    h = _dot(h, w2[...]) + b2[...]
    o_ref[...] = _ln(h, g[...], bln[...])


def _enc_pre_body(x_ref, w0, b0, w1, b1, w2, b2, g, bln, ws, wr,
                  o_ref, hs_ref, hr_ref):
    h = jnp.maximum(_dot(x_ref[...], w0[...]) + b0[...], 0.0)
    h = jnp.maximum(_dot(h, w1[...]) + b1[...], 0.0)
    h = _dot(h, w2[...]) + b2[...]
    out = _ln(h, g[...], bln[...])
    o_ref[...] = out
    hs_ref[...] = _dot(out, ws[...])
    hr_ref[...] = _dot(out, wr[...])


def _edge_body(g_ref, e_ref, we0, b0, w1, b1, w2, b2, g, bln,
               enew_ref, eout_ref):
    e = e_ref[...]
    h = jnp.maximum(g_ref[...] + _dot(e, we0[...]) + b0[...], 0.0)
    h = jnp.maximum(_dot(h, w1[...]) + b1[...], 0.0)
    h = _dot(h, w2[...]) + b2[...]
    enew = _ln(h, g[...], bln[...])
    enew_ref[...] = enew
    eout_ref[...] = e + enew


def _edge_last_body(g_ref, e_ref, we0, b0, w1, b1, w2, b2, g, bln,
                    enew_ref):
    e = e_ref[...]
    h = jnp.maximum(g_ref[...] + _dot(e, we0[...]) + b0[...], 0.0)
    h = jnp.maximum(_dot(h, w1[...]) + b1[...], 0.0)
    h = _dot(h, w2[...]) + b2[...]
    enew_ref[...] = _ln(h, g[...], bln[...])


def _node_body(h_ref, a0_ref, a1_ref, w0h, w0a, b0, w1, b1, w2, b2, g, bln,
               hout_ref):
    hin = h_ref[...]
    a = a0_ref[...] + a1_ref[...]
    h = jnp.maximum(_dot(hin, w0h[...]) + _dot(a, w0a[...]) + b0[...], 0.0)
    h = jnp.maximum(_dot(h, w1[...]) + b1[...], 0.0)
    h = _dot(h, w2[...]) + b2[...]
    hout_ref[...] = hin + _ln(h, g[...], bln[...])


def _node_pre_body(h_ref, a0_ref, a1_ref, w0h, w0a, b0, w1, b1, w2, b2,
                   g, bln, ws, wr, hout_ref, hs_ref, hr_ref):
    hin = h_ref[...]
    a = a0_ref[...] + a1_ref[...]
    h = jnp.maximum(_dot(hin, w0h[...]) + _dot(a, w0a[...]) + b0[...], 0.0)
    h = jnp.maximum(_dot(h, w1[...]) + b1[...], 0.0)
    h = _dot(h, w2[...]) + b2[...]
    hout = hin + _ln(h, g[...], bln[...])
    hout_ref[...] = hout
    hs_ref[...] = _dot(hout, ws[...])
    hr_ref[...] = _dot(hout, wr[...])


def _dec_body(h_ref, f_ref, w0, b0, w1, b1, w2, b2, std, mean, o_ref):
    h = jnp.maximum(_dot(h_ref[...], w0[...]) + b0[...], 0.0)
    h = jnp.maximum(_dot(h, w1[...]) + b1[...], 0.0)
    d = _dot(h, w2[...]) + b2[...]
    o_ref[...] = f_ref[...] + d * std[...] + mean[...]


def _row_spec(n_rows, b, k):
    return pl.BlockSpec((b, k), lambda i: (i, 0))


def _enc_call(x, w0, b0, w1, b1, w2, b2, g, bln, b_rows):
    n, k = x.shape
    grid = n // b_rows
    return pl.pallas_call(
        _enc_body,
        grid=(grid,),
        in_specs=[_row_spec(n, b_rows, k)] + [_full(w.shape) for w in
                  (w0, b0, w1, b1, w2, b2, g, bln)],
        out_specs=_row_spec(n, b_rows, H),
        out_shape=jax.ShapeDtypeStruct((n, H), jnp.float32),
    )(x, w0, b0, w1, b1, w2, b2, g, bln)


def _enc_pre_call(x, w0, b0, w1, b1, w2, b2, g, bln, ws, wr):
    n, k = x.shape
    grid = n // B_NODE
    spec = _row_spec(n, B_NODE, H)
    return pl.pallas_call(
        _enc_pre_body,
        grid=(grid,),
        in_specs=[_row_spec(n, B_NODE, k)] + [_full(w.shape) for w in
                  (w0, b0, w1, b1, w2, b2, g, bln, ws, wr)],
        out_specs=[spec, spec, spec],
        out_shape=[jax.ShapeDtypeStruct((n, H), jnp.float32)] * 3,
    )(x, w0, b0, w1, b1, w2, b2, g, bln, ws, wr)


def _edge_call(gsum, e, we0, b0, w1, b1, w2, b2, g, bln):
    grid = N_EDGES // B_EDGE
    spec = _row_spec(N_EDGES, B_EDGE, H)
    return pl.pallas_call(
        _edge_body,
        grid=(grid,),
        in_specs=[spec, spec] + [_full(w.shape) for w in
                  (we0, b0, w1, b1, w2, b2, g, bln)],
        out_specs=[spec, spec],
        out_shape=[jax.ShapeDtypeStruct((N_EDGES, H), jnp.float32)] * 2,
    )(gsum, e, we0, b0, w1, b1, w2, b2, g, bln)


def _edge_last_call(gsum, e, we0, b0, w1, b1, w2, b2, g, bln):
    grid = N_EDGES // B_EDGE
    spec = _row_spec(N_EDGES, B_EDGE, H)
    return pl.pallas_call(
        _edge_last_body,
        grid=(grid,),
        in_specs=[spec, spec] + [_full(w.shape) for w in
                  (we0, b0, w1, b1, w2, b2, g, bln)],
        out_specs=spec,
        out_shape=jax.ShapeDtypeStruct((N_EDGES, H), jnp.float32),
    )(gsum, e, we0, b0, w1, b1, w2, b2, g, bln)


def _node_call(h, a0, a1, w0h, w0a, b0, w1, b1, w2, b2, g, bln):
    grid = N_NODES // B_NODE
    spec = _row_spec(N_NODES, B_NODE, H)
    return pl.pallas_call(
        _node_body,
        grid=(grid,),
        in_specs=[spec, spec, spec] + [_full(w.shape) for w in
                  (w0h, w0a, b0, w1, b1, w2, b2, g, bln)],
        out_specs=spec,
        out_shape=jax.ShapeDtypeStruct((N_NODES, H), jnp.float32),
    )(h, a0, a1, w0h, w0a, b0, w1, b1, w2, b2, g, bln)


def _node_pre_call(h, a0, a1, w0h, w0a, b0, w1, b1, w2, b2, g, bln, ws, wr):
    grid = N_NODES // B_NODE
    spec = _row_spec(N_NODES, B_NODE, H)
    return pl.pallas_call(
        _node_pre_body,
        grid=(grid,),
        in_specs=[spec, spec, spec] + [_full(w.shape) for w in
                  (w0h, w0a, b0, w1, b1, w2, b2, g, bln, ws, wr)],
        out_specs=[spec, spec, spec],
        out_shape=[jax.ShapeDtypeStruct((N_NODES, H), jnp.float32)] * 3,
    )(h, a0, a1, w0h, w0a, b0, w1, b1, w2, b2, g, bln, ws, wr)


def _dec_call(h, frames_p, w0, b0, w1, b1, w2, b2, std, mean):
    grid = N_NODES // B_NODE
    return pl.pallas_call(
        _dec_body,
        grid=(grid,),
        in_specs=[_row_spec(N_NODES, B_NODE, H),
                  _row_spec(N_NODES, B_NODE, 8)] +
                 [_full(w.shape) for w in (w0, b0, w1, b1, w2, b2, std, mean)],
        out_specs=_row_spec(N_NODES, B_NODE, 8),
        out_shape=jax.ShapeDtypeStruct((N_NODES, 8), jnp.float32),
    )(h, frames_p, w0, b0, w1, b1, w2, b2, std, mean)


def _sc_mesh():
    return plsc.VectorSubcoreMesh(core_axis_name="c", subcore_axis_name="s",
                                  num_cores=NC, num_subcores=NS)


@functools.cache
def _gather_sum_kernel():
    @functools.partial(
        pl.kernel,
        out_type=jax.ShapeDtypeStruct((N_EDGES, H), jnp.float32),
        mesh=_sc_mesh(),
        scratch_types=[
            pltpu.VMEM((BPW,), jnp.int32),
            pltpu.VMEM((BPW,), jnp.int32),
            pltpu.VMEM((CG, H), jnp.float32),
            pltpu.VMEM((CG, H), jnp.float32),
            pltpu.VMEM((CG, H), jnp.float32),
            pltpu.VMEM((CG, H), jnp.float32),
            pltpu.VMEM((CG, H), jnp.float32),
            pltpu.VMEM((CG, H), jnp.float32),
            pltpu.SemaphoreType.DMA,
            pltpu.SemaphoreType.DMA,
            pltpu.SemaphoreType.DMA,
            pltpu.SemaphoreType.DMA,
            pltpu.SemaphoreType.DMA,
            pltpu.SemaphoreType.DMA,
        ],
    )
    def gather_sum(hs_hbm, hr_hbm, s_hbm, r_hbm, out_hbm,
                   sidx, ridx, srows0, rrows0, srows1, rrows1,
                   srows2, rrows2, ss0, sr0, ss1, sr1, ss2, sr2):
        cid = lax.axis_index("c")
        sid = lax.axis_index("s")
        base = (sid * NC + cid) * BPW
        pltpu.sync_copy(s_hbm.at[pl.ds(base, BPW)], sidx)
        pltpu.sync_copy(r_hbm.at[pl.ds(base, BPW)], ridx)
        bufs = ((srows0, rrows0, ss0, sr0), (srows1, rrows1, ss1, sr1),
                (srows2, rrows2, ss2, sr2))

        def issue(off, b):
            sb, rb, ss, sr = bufs[b]
            pltpu.async_copy(hs_hbm.at[sidx.at[pl.ds(off, CG)]], sb, ss)
            pltpu.async_copy(hr_hbm.at[ridx.at[pl.ds(off, CG)]], rb, sr)

        def drain_compute(off, b):
            sb, rb, ss, sr = bufs[b]
            pltpu.make_async_copy(hs_hbm.at[sidx.at[pl.ds(off, CG)]], sb,
                                  ss).wait()
            pltpu.make_async_copy(hr_hbm.at[ridx.at[pl.ds(off, CG)]], rb,
                                  sr).wait()

            def row(i, c):
                for j in range(H // 16):
                    sl = pl.ds(j * 16, 16)
                    sb[i, sl] = sb[i, sl] + rb[i, sl]
                return c
            lax.fori_loop(0, CG, row, 0)
            pltpu.sync_copy(sb, out_hbm.at[pl.ds(base + off, CG)])

        for b in range(3):
            issue(b * CG, b)

        def round_(g, c):
            for b in range(3):
                k = g * 3 + b
                drain_compute(k * CG, b)
                issue((k + 3) * CG, b)
            return c
        lax.fori_loop(0, NCH // 3 - 1, round_, 0)
        for b in range(3):
            drain_compute((NCH - 3 + b) * CG, b)

        sb, rb, ss, _ = bufs[1]
        toff = NCH * CG
        pltpu.async_copy(hs_hbm.at[sidx.at[pl.ds(toff, TAIL)]],
                         sb.at[pl.ds(0, TAIL)], ss).wait()
        pltpu.async_copy(hr_hbm.at[ridx.at[pl.ds(toff, TAIL)]],
                         rb.at[pl.ds(0, TAIL)], ss).wait()

        def trow(i, c):
            for j in range(H // 16):
                sl = pl.ds(j * 16, 16)
                sb[i, sl] = sb[i, sl] + rb[i, sl]
            return c
        lax.fori_loop(0, TAIL, trow, 0)
        pltpu.sync_copy(sb.at[pl.ds(0, TAIL)],
                        out_hbm.at[pl.ds(base + toff, TAIL)])

    return gather_sum


@functools.cache
def _scatter_add_kernel():
    @functools.partial(
        pl.kernel,
        out_type=(jax.ShapeDtypeStruct((N_NODES, H), jnp.float32),
                  jax.ShapeDtypeStruct((N_NODES, H), jnp.float32)),
        mesh=_sc_mesh(),
        scratch_types=[
            pltpu.VMEM_SHARED((N_NODES, H), jnp.float32),
            pltpu.VMEM((CG,), jnp.int32),
            pltpu.VMEM((CG,), jnp.int32),
            pltpu.VMEM((CG,), jnp.int32),
            pltpu.VMEM((TAIL,), jnp.int32),
            pltpu.VMEM((CG, H), jnp.float32),
            pltpu.VMEM((CG, H), jnp.float32),
            pltpu.VMEM((CG, H), jnp.float32),
            pltpu.SemaphoreType.DMA,
            pltpu.SemaphoreType.DMA,
            pltpu.SemaphoreType.DMA,
            pltpu.SemaphoreType.DMA,
            pltpu.SemaphoreType.DMA,
            pltpu.SemaphoreType.DMA,
        ],
    )
    def scatter_add(enew_hbm, r_hbm, zeros_hbm, out0, out1,
                    acc, idxc0, idxc1, idxc2, idxt, rows0, rows1, rows2,
                    si0, sd0, si1, sd1, si2, sd2):
        cid = lax.axis_index("c")
        sid = lax.axis_index("s")
        nsl = pl.ds(sid * NPT, NPT)
        nsl_last = pl.ds((NS - 1) * NPT, NPT_LAST)

        @pl.when(sid < NS - 1)
        def _():
            pltpu.sync_copy(zeros_hbm.at[nsl], acc.at[nsl])

        @pl.when(sid == NS - 1)
        def _():
            pltpu.sync_copy(zeros_hbm.at[nsl_last], acc.at[nsl_last])
        plsc.subcore_barrier()

        base = cid * (N_EDGES // NC) + sid * BPW
        bufs = ((idxc0, rows0, si0, sd0), (idxc1, rows1, si1, sd1),
                (idxc2, rows2, si2, sd2))

        def issue(off, b):
            ib, rb, si, sd = bufs[b]
            pltpu.async_copy(r_hbm.at[pl.ds(base + off, CG)], ib, si)
            pltpu.async_copy(enew_hbm.at[pl.ds(base + off, CG)], rb, sd)

        def drain_scatter(off, b):
            ib, rb, si, sd = bufs[b]
            pltpu.make_async_copy(r_hbm.at[pl.ds(base + off, CG)], ib,
                                  si).wait()
            pltpu.make_async_copy(enew_hbm.at[pl.ds(base + off, CG)], rb,
                                  sd).wait()
            pltpu.sync_copy(rb, acc.at[ib], add=True)

        for b in range(3):
            issue(b * CG, b)

        def round_(g, c):
            for b in range(3):
                k = g * 3 + b
                drain_scatter(k * CG, b)
                issue((k + 3) * CG, b)
            return c
        lax.fori_loop(0, NCH // 3 - 1, round_, 0)
        for b in range(3):
            drain_scatter((NCH - 3 + b) * CG, b)

        toff = NCH * CG
        pltpu.sync_copy(r_hbm.at[pl.ds(base + toff, TAIL)], idxt)
        pltpu.sync_copy(enew_hbm.at[pl.ds(base + toff, TAIL)],
                        rows1.at[pl.ds(0, TAIL)])
        pltpu.sync_copy(rows1.at[pl.ds(0, TAIL)], acc.at[idxt], add=True)
        plsc.subcore_barrier()

        @pl.when((cid == 0) & (sid < NS - 1))
        def _():
            pltpu.sync_copy(acc.at[nsl], out0.at[nsl])

        @pl.when((cid == 0) & (sid == NS - 1))
        def _():
            pltpu.sync_copy(acc.at[nsl_last], out0.at[nsl_last])

        @pl.when((cid == 1) & (sid < NS - 1))
        def _():
            pltpu.sync_copy(acc.at[nsl], out1.at[nsl])

        @pl.when((cid == 1) & (sid == NS - 1))
        def _():
            pltpu.sync_copy(acc.at[nsl_last], out1.at[nsl_last])

    return scatter_add


def _gather_sum(hs, hr, senders, receivers):
    return _gather_sum_kernel()(hs, hr, senders, receivers)


def _scatter_add(e_new, receivers, zeros):
    return _scatter_add_kernel()(e_new, receivers, zeros)


def _r2(b):
    return b.reshape(1, -1)


def kernel(x, edge_index, edge_attr, velocity_sequence_noise, params):
    del velocity_sequence_noise
    frames = x[:, 1:3]
    node_type = x[:, 0].astype(jnp.int32)
    one_hot = jax.nn.one_hot(node_type, 9, dtype=jnp.float32)
    node_feats = jnp.concatenate([frames, one_hot], axis=1)
    nn = params["node_norm"]
    node_attr = (node_feats - nn["mean"]) / nn["std"]
    node_attr_p = jnp.pad(node_attr, ((0, 0), (0, 5)))
    edge_attr_p = jnp.pad(edge_attr, ((0, 0), (0, 4)))

    blocks = params["blocks"]
    splits = [blk["eb"]["l0"]["W"] for blk in blocks]
    enb, eeb = params["enc_nb"], params["enc_eb"]
    h, hs, hr = _enc_pre_call(
        node_attr_p,
        jnp.pad(enb["l0"]["W"], ((0, 5), (0, 0))), _r2(enb["l0"]["b"]),
        enb["l1"]["W"], _r2(enb["l1"]["b"]),
        enb["l2"]["W"], _r2(enb["l2"]["b"]),
        _r2(enb["ln"]["g"]), _r2(enb["ln"]["b"]),
        splits[0][:H], splits[0][H:2 * H])
    e = _enc_call(edge_attr_p,
                  jnp.pad(eeb["l0"]["W"], ((0, 4), (0, 0))), _r2(eeb["l0"]["b"]),
                  eeb["l1"]["W"], _r2(eeb["l1"]["b"]),
                  eeb["l2"]["W"], _r2(eeb["l2"]["b"]),
                  _r2(eeb["ln"]["g"]), _r2(eeb["ln"]["b"]), B_EDGE)

    senders = edge_index[0]
    receivers = edge_index[1]
    zeros = jnp.zeros((N_NODES, H), jnp.float32)

    for k, blk in enumerate(blocks):
        eb, nb = blk["eb"], blk["nb"]
        last = k == len(blocks) - 1
        we = splits[k][2 * H:]
        gsum = _gather_sum(hs, hr, senders, receivers)
        eargs = (gsum, e, we, _r2(eb["l0"]["b"]),
                 eb["l1"]["W"], _r2(eb["l1"]["b"]),
                 eb["l2"]["W"], _r2(eb["l2"]["b"]),
                 _r2(eb["ln"]["g"]), _r2(eb["ln"]["b"]))
        if last:
            e_new = _edge_last_call(*eargs)
        else:
            e_new, e = _edge_call(*eargs)
        a0, a1 = _scatter_add(e_new, receivers, zeros)
        n0 = nb["l0"]["W"]
        nargs = (h, a0, a1, n0[:H], n0[H:], _r2(nb["l0"]["b"]),
                 nb["l1"]["W"], _r2(nb["l1"]["b"]),
                 nb["l2"]["W"], _r2(nb["l2"]["b"]),
                 _r2(nb["ln"]["g"]), _r2(nb["ln"]["b"]))
        if last:
            h = _node_call(*nargs)
        else:
            h, hs, hr = _node_pre_call(*nargs, splits[k + 1][:H],
                                       splits[k + 1][H:2 * H])

    dec = params["dec"]
    on = params["out_norm"]
    frames_p = jnp.pad(frames, ((0, 0), (0, 6)))
    w2p = jnp.pad(dec["l2"]["W"], ((0, 0), (0, 6)))
    b2p = jnp.pad(dec["l2"]["b"], (0, 6))
    stdp = jnp.pad(on["std"], (0, 6), constant_values=1.0)
    meanp = jnp.pad(on["mean"], (0, 6))
    out = _dec_call(h, frames_p,
                    dec["l0"]["W"], _r2(dec["l0"]["b"]),
                    dec["l1"]["W"], _r2(dec["l1"]["b"]),
                    w2p, _r2(b2p), _r2(stdp), _r2(meanp))
    return out[:, :2]


# speedup vs baseline: 4.7183x; 1.0007x over previous
import functools

import jax
import jax.numpy as jnp
from jax import lax
from jax.experimental import pallas as pl
from jax.experimental.pallas import tpu as pltpu
from jax.experimental.pallas import tpu_sc as plsc

N_NODES = 10000
N_EDGES = 160000
H = 128

B_NODE = 2000
B_EDGE = 10000

NC = 2
NS = 16
NW = NC * NS
BPW = N_EDGES // NW
CG = 128
NCH = BPW // CG
TAIL = BPW - NCH * CG
NPT = 632
NPT_LAST = N_NODES - (NS - 1) * NPT


def _ln(h, g, b):
    mu = jnp.mean(h, axis=-1, keepdims=True)
    var = jnp.mean((h - mu) * (h - mu), axis=-1, keepdims=True)
    return (h - mu) * lax.rsqrt(var + 1e-5) * g + b


def _dot(a, b):
    return jnp.dot(a, b, preferred_element_type=jnp.float32)


def _full(shape):
    return pl.BlockSpec(shape, lambda i: (0,) * len(shape))


def _enc_body(x_ref, w0, b0, w1, b1, w2, b2, g, bln, o_ref):
    h = jnp.maximum(_dot(x_ref[...], w0[...]) + b0[...], 0.0)
    h = jnp.maximum(_dot(h, w1[...]) + b1[...], 0.0)
    h = _dot(h, w2[...]) + b2[...]
    o_ref[...] = _ln(h, g[...], bln[...])


def _enc_pre_body(x_ref, w0, b0, w1, b1, w2, b2, g, bln, ws, wr,
                  o_ref, hs_ref, hr_ref):
    h = jnp.maximum(_dot(x_ref[...], w0[...]) + b0[...], 0.0)
    h = jnp.maximum(_dot(h, w1[...]) + b1[...], 0.0)
    h = _dot(h, w2[...]) + b2[...]
    out = _ln(h, g[...], bln[...])
    o_ref[...] = out
    hs_ref[...] = _dot(out, ws[...])
    hr_ref[...] = _dot(out, wr[...])


def _edge_body(g_ref, e_ref, we0, b0, w1, b1, w2, b2, g, bln,
               enew_ref, eout_ref):
    e = e_ref[...]
    h = jnp.maximum(g_ref[...] + _dot(e, we0[...]) + b0[...], 0.0)
    h = jnp.maximum(_dot(h, w1[...]) + b1[...], 0.0)
    h = _dot(h, w2[...]) + b2[...]
    enew = _ln(h, g[...], bln[...])
    enew_ref[...] = enew
    eout_ref[...] = e + enew


def _edge_last_body(g_ref, e_ref, we0, b0, w1, b1, w2, b2, g, bln,
                    enew_ref):
    e = e_ref[...]
    h = jnp.maximum(g_ref[...] + _dot(e, we0[...]) + b0[...], 0.0)
    h = jnp.maximum(_dot(h, w1[...]) + b1[...], 0.0)
    h = _dot(h, w2[...]) + b2[...]
    enew_ref[...] = _ln(h, g[...], bln[...])


def _node_body(h_ref, a0_ref, a1_ref, w0h, w0a, b0, w1, b1, w2, b2, g, bln,
               hout_ref):
    hin = h_ref[...]
    a = a0_ref[...] + a1_ref[...]
    h = jnp.maximum(_dot(hin, w0h[...]) + _dot(a, w0a[...]) + b0[...], 0.0)
    h = jnp.maximum(_dot(h, w1[...]) + b1[...], 0.0)
    h = _dot(h, w2[...]) + b2[...]
    hout_ref[...] = hin + _ln(h, g[...], bln[...])


def _node_pre_body(h_ref, a0_ref, a1_ref, w0h, w0a, b0, w1, b1, w2, b2,
                   g, bln, ws, wr, hout_ref, hs_ref, hr_ref):
    hin = h_ref[...]
    a = a0_ref[...] + a1_ref[...]
    h = jnp.maximum(_dot(hin, w0h[...]) + _dot(a, w0a[...]) + b0[...], 0.0)
    h = jnp.maximum(_dot(h, w1[...]) + b1[...], 0.0)
    h = _dot(h, w2[...]) + b2[...]
    hout = hin + _ln(h, g[...], bln[...])
    hout_ref[...] = hout
    hs_ref[...] = _dot(hout, ws[...])
    hr_ref[...] = _dot(hout, wr[...])


def _dec_body(h_ref, f_ref, w0, b0, w1, b1, w2, b2, std, mean, o_ref):
    h = jnp.maximum(_dot(h_ref[...], w0[...]) + b0[...], 0.0)
    h = jnp.maximum(_dot(h, w1[...]) + b1[...], 0.0)
    d = _dot(h, w2[...]) + b2[...]
    o_ref[...] = f_ref[...] + d * std[...] + mean[...]


def _row_spec(n_rows, b, k):
    return pl.BlockSpec((b, k), lambda i: (i, 0))


def _enc_call(x, w0, b0, w1, b1, w2, b2, g, bln, b_rows):
    n, k = x.shape
    grid = n // b_rows
    return pl.pallas_call(
        _enc_body,
        grid=(grid,),
        in_specs=[_row_spec(n, b_rows, k)] + [_full(w.shape) for w in
                  (w0, b0, w1, b1, w2, b2, g, bln)],
        out_specs=_row_spec(n, b_rows, H),
        out_shape=jax.ShapeDtypeStruct((n, H), jnp.float32),
    )(x, w0, b0, w1, b1, w2, b2, g, bln)


def _enc_pre_call(x, w0, b0, w1, b1, w2, b2, g, bln, ws, wr):
    n, k = x.shape
    grid = n // B_NODE
    spec = _row_spec(n, B_NODE, H)
    return pl.pallas_call(
        _enc_pre_body,
        grid=(grid,),
        in_specs=[_row_spec(n, B_NODE, k)] + [_full(w.shape) for w in
                  (w0, b0, w1, b1, w2, b2, g, bln, ws, wr)],
        out_specs=[spec, spec, spec],
        out_shape=[jax.ShapeDtypeStruct((n, H), jnp.float32)] * 3,
    )(x, w0, b0, w1, b1, w2, b2, g, bln, ws, wr)


def _edge_call(gsum, e, we0, b0, w1, b1, w2, b2, g, bln):
    grid = N_EDGES // B_EDGE
    spec = _row_spec(N_EDGES, B_EDGE, H)
    return pl.pallas_call(
        _edge_body,
        grid=(grid,),
        in_specs=[spec, spec] + [_full(w.shape) for w in
                  (we0, b0, w1, b1, w2, b2, g, bln)],
        out_specs=[spec, spec],
        out_shape=[jax.ShapeDtypeStruct((N_EDGES, H), jnp.float32)] * 2,
    )(gsum, e, we0, b0, w1, b1, w2, b2, g, bln)


def _edge_last_call(gsum, e, we0, b0, w1, b1, w2, b2, g, bln):
    grid = N_EDGES // B_EDGE
    spec = _row_spec(N_EDGES, B_EDGE, H)
    return pl.pallas_call(
        _edge_last_body,
        grid=(grid,),
        in_specs=[spec, spec] + [_full(w.shape) for w in
                  (we0, b0, w1, b1, w2, b2, g, bln)],
        out_specs=spec,
        out_shape=jax.ShapeDtypeStruct((N_EDGES, H), jnp.float32),
    )(gsum, e, we0, b0, w1, b1, w2, b2, g, bln)


def _node_call(h, a0, a1, w0h, w0a, b0, w1, b1, w2, b2, g, bln):
    grid = N_NODES // B_NODE
    spec = _row_spec(N_NODES, B_NODE, H)
    return pl.pallas_call(
        _node_body,
        grid=(grid,),
        in_specs=[spec, spec, spec] + [_full(w.shape) for w in
                  (w0h, w0a, b0, w1, b1, w2, b2, g, bln)],
        out_specs=spec,
        out_shape=jax.ShapeDtypeStruct((N_NODES, H), jnp.float32),
    )(h, a0, a1, w0h, w0a, b0, w1, b1, w2, b2, g, bln)


def _node_pre_call(h, a0, a1, w0h, w0a, b0, w1, b1, w2, b2, g, bln, ws, wr):
    grid = N_NODES // B_NODE
    spec = _row_spec(N_NODES, B_NODE, H)
    return pl.pallas_call(
        _node_pre_body,
        grid=(grid,),
        in_specs=[spec, spec, spec] + [_full(w.shape) for w in
                  (w0h, w0a, b0, w1, b1, w2, b2, g, bln, ws, wr)],
        out_specs=[spec, spec, spec],
        out_shape=[jax.ShapeDtypeStruct((N_NODES, H), jnp.float32)] * 3,
    )(h, a0, a1, w0h, w0a, b0, w1, b1, w2, b2, g, bln, ws, wr)


def _dec_call(h, frames_p, w0, b0, w1, b1, w2, b2, std, mean):
    grid = N_NODES // B_NODE
    return pl.pallas_call(
        _dec_body,
        grid=(grid,),
        in_specs=[_row_spec(N_NODES, B_NODE, H),
                  _row_spec(N_NODES, B_NODE, 8)] +
                 [_full(w.shape) for w in (w0, b0, w1, b1, w2, b2, std, mean)],
        out_specs=_row_spec(N_NODES, B_NODE, 8),
        out_shape=jax.ShapeDtypeStruct((N_NODES, 8), jnp.float32),
    )(h, frames_p, w0, b0, w1, b1, w2, b2, std, mean)


def _sc_mesh():
    return plsc.VectorSubcoreMesh(core_axis_name="c", subcore_axis_name="s",
                                  num_cores=NC, num_subcores=NS)


@functools.cache
def _gather_sum_kernel():
    @functools.partial(
        pl.kernel,
        out_type=jax.ShapeDtypeStruct((N_EDGES, H), jnp.float32),
        mesh=_sc_mesh(),
        scratch_types=[
            pltpu.VMEM((BPW,), jnp.int32),
            pltpu.VMEM((BPW,), jnp.int32),
            pltpu.VMEM((CG, H), jnp.float32),
            pltpu.VMEM((CG, H), jnp.float32),
            pltpu.VMEM((CG, H), jnp.float32),
            pltpu.VMEM((CG, H), jnp.float32),
            pltpu.VMEM((CG, H), jnp.float32),
            pltpu.VMEM((CG, H), jnp.float32),
            pltpu.SemaphoreType.DMA,
            pltpu.SemaphoreType.DMA,
            pltpu.SemaphoreType.DMA,
            pltpu.SemaphoreType.DMA,
            pltpu.SemaphoreType.DMA,
            pltpu.SemaphoreType.DMA,
        ],
    )
    def gather_sum(hs_hbm, hr_hbm, s_hbm, r_hbm, out_hbm,
                   sidx, ridx, srows0, rrows0, srows1, rrows1,
                   srows2, rrows2, ss0, sr0, ss1, sr1, ss2, sr2):
        cid = lax.axis_index("c")
        sid = lax.axis_index("s")
        base = (sid * NC + cid) * BPW
        pltpu.sync_copy(s_hbm.at[pl.ds(base, BPW)], sidx)
        pltpu.sync_copy(r_hbm.at[pl.ds(base, BPW)], ridx)
        bufs = ((srows0, rrows0, ss0, sr0), (srows1, rrows1, ss1, sr1),
                (srows2, rrows2, ss2, sr2))

        def issue(off, b):
            sb, rb, ss, sr = bufs[b]
            pltpu.async_copy(hs_hbm.at[sidx.at[pl.ds(off, CG)]], sb, ss)
            pltpu.async_copy(hr_hbm.at[ridx.at[pl.ds(off, CG)]], rb, sr)

        def drain_compute(off, b):
            sb, rb, ss, sr = bufs[b]
            pltpu.make_async_copy(hs_hbm.at[sidx.at[pl.ds(off, CG)]], sb,
                                  ss).wait()
            pltpu.make_async_copy(hr_hbm.at[ridx.at[pl.ds(off, CG)]], rb,
                                  sr).wait()

            def row(i, c):
                for j in range(H // 16):
                    sl = pl.ds(j * 16, 16)
                    sb[i, sl] = sb[i, sl] + rb[i, sl]
                return c
            lax.fori_loop(0, CG, row, 0)
            pltpu.sync_copy(sb, out_hbm.at[pl.ds(base + off, CG)])

        for b in range(3):
            issue(b * CG, b)

        def round_(g, c):
            for b in range(3):
                k = g * 3 + b
                drain_compute(k * CG, b)
                issue((k + 3) * CG, b)
            return c
        lax.fori_loop(0, NCH // 3 - 1, round_, 0)
        for b in range(3):
            drain_compute((NCH - 3 + b) * CG, b)

        sb, rb, ss, _ = bufs[1]
        toff = NCH * CG
        pltpu.async_copy(hs_hbm.at[sidx.at[pl.ds(toff, TAIL)]],
                         sb.at[pl.ds(0, TAIL)], ss).wait()
        pltpu.async_copy(hr_hbm.at[ridx.at[pl.ds(toff, TAIL)]],
                         rb.at[pl.ds(0, TAIL)], ss).wait()

        def trow(i, c):
            for j in range(H // 16):
                sl = pl.ds(j * 16, 16)
                sb[i, sl] = sb[i, sl] + rb[i, sl]
            return c
        lax.fori_loop(0, TAIL, trow, 0)
        pltpu.sync_copy(sb.at[pl.ds(0, TAIL)],
                        out_hbm.at[pl.ds(base + toff, TAIL)])

    return gather_sum


@functools.cache
def _scatter_add_kernel():
    @functools.partial(
        pl.kernel,
        out_type=(jax.ShapeDtypeStruct((N_NODES, H), jnp.float32),
                  jax.ShapeDtypeStruct((N_NODES, H), jnp.float32)),
        mesh=_sc_mesh(),
        scratch_types=[
            pltpu.VMEM_SHARED((N_NODES, H), jnp.float32),
            pltpu.VMEM((CG,), jnp.int32),
            pltpu.VMEM((CG,), jnp.int32),
            pltpu.VMEM((CG,), jnp.int32),
            pltpu.VMEM((TAIL,), jnp.int32),
            pltpu.VMEM((CG, H), jnp.float32),
            pltpu.VMEM((CG, H), jnp.float32),
            pltpu.VMEM((CG, H), jnp.float32),
            pltpu.SemaphoreType.DMA,
            pltpu.SemaphoreType.DMA,
            pltpu.SemaphoreType.DMA,
            pltpu.SemaphoreType.DMA,
            pltpu.SemaphoreType.DMA,
            pltpu.SemaphoreType.DMA,
        ],
    )
    def scatter_add(enew_hbm, r_hbm, zeros_hbm, out0, out1,
                    acc, idxc0, idxc1, idxc2, idxt, rows0, rows1, rows2,
                    si0, sd0, si1, sd1, si2, sd2):
        cid = lax.axis_index("c")
        sid = lax.axis_index("s")
        nsl = pl.ds(sid * NPT, NPT)
        nsl_last = pl.ds((NS - 1) * NPT, NPT_LAST)

        @pl.when(sid < NS - 1)
        def _():
            pltpu.sync_copy(zeros_hbm.at[nsl], acc.at[nsl])

        @pl.when(sid == NS - 1)
        def _():
            pltpu.sync_copy(zeros_hbm.at[nsl_last], acc.at[nsl_last])
        plsc.subcore_barrier()

        base = cid * (N_EDGES // NC) + sid * BPW
        bufs = ((idxc0, rows0, si0, sd0), (idxc1, rows1, si1, sd1),
                (idxc2, rows2, si2, sd2))

        def issue(off, b):
            ib, rb, si, sd = bufs[b]
            pltpu.async_copy(r_hbm.at[pl.ds(base + off, CG)], ib, si)
            pltpu.async_copy(enew_hbm.at[pl.ds(base + off, CG)], rb, sd)

        def drain_scatter(off, b):
            ib, rb, si, sd = bufs[b]
            pltpu.make_async_copy(r_hbm.at[pl.ds(base + off, CG)], ib,
                                  si).wait()
            pltpu.make_async_copy(enew_hbm.at[pl.ds(base + off, CG)], rb,
                                  sd).wait()
            pltpu.sync_copy(rb, acc.at[ib], add=True)

        for b in range(3):
            issue(b * CG, b)

        def round_(g, c):
            for b in range(3):
                k = g * 3 + b
                drain_scatter(k * CG, b)
                issue((k + 3) * CG, b)
            return c
        lax.fori_loop(0, NCH // 3 - 1, round_, 0)
        for b in range(3):
            drain_scatter((NCH - 3 + b) * CG, b)

        toff = NCH * CG
        pltpu.sync_copy(r_hbm.at[pl.ds(base + toff, TAIL)], idxt)
        pltpu.sync_copy(enew_hbm.at[pl.ds(base + toff, TAIL)],
                        rows1.at[pl.ds(0, TAIL)])
        pltpu.sync_copy(rows1.at[pl.ds(0, TAIL)], acc.at[idxt], add=True)
        plsc.subcore_barrier()

        @pl.when((cid == 0) & (sid < NS - 1))
        def _():
            pltpu.sync_copy(acc.at[nsl], out0.at[nsl])

        @pl.when((cid == 0) & (sid == NS - 1))
        def _():
            pltpu.sync_copy(acc.at[nsl_last], out0.at[nsl_last])

        @pl.when((cid == 1) & (sid < NS - 1))
        def _():
            pltpu.sync_copy(acc.at[nsl], out1.at[nsl])

        @pl.when((cid == 1) & (sid == NS - 1))
        def _():
            pltpu.sync_copy(acc.at[nsl_last], out1.at[nsl_last])

    return scatter_add


def _gather_sum(hs, hr, senders, receivers):
    return _gather_sum_kernel()(hs, hr, senders, receivers)


def _scatter_add(e_new, receivers, zeros):
    return _scatter_add_kernel()(e_new, receivers, zeros)


def _r2(b):
    return b.reshape(1, -1)


def kernel(x, edge_index, edge_attr, velocity_sequence_noise, params):
    del velocity_sequence_noise
    frames = x[:, 1:3]
    node_type = x[:, 0].astype(jnp.int32)
    one_hot = jax.nn.one_hot(node_type, 9, dtype=jnp.float32)
    node_feats = jnp.concatenate([frames, one_hot], axis=1)
    nn = params["node_norm"]
    node_attr = (node_feats - nn["mean"]) / nn["std"]
    node_attr_p = jnp.pad(node_attr, ((0, 0), (0, 5)))
    edge_attr_p = jnp.pad(edge_attr, ((0, 0), (0, 4)))

    blocks = params["blocks"]
    splits = [blk["eb"]["l0"]["W"] for blk in blocks]
    enb, eeb = params["enc_nb"], params["enc_eb"]
    h, hs, hr = _enc_pre_call(
        node_attr_p,
        jnp.pad(enb["l0"]["W"], ((0, 5), (0, 0))), _r2(enb["l0"]["b"]),
        enb["l1"]["W"], _r2(enb["l1"]["b"]),
        enb["l2"]["W"], _r2(enb["l2"]["b"]),
        _r2(enb["ln"]["g"]), _r2(enb["ln"]["b"]),
        splits[0][:H], splits[0][H:2 * H])
    e = _enc_call(edge_attr_p,
                  jnp.pad(eeb["l0"]["W"], ((0, 4), (0, 0))), _r2(eeb["l0"]["b"]),
                  eeb["l1"]["W"], _r2(eeb["l1"]["b"]),
                  eeb["l2"]["W"], _r2(eeb["l2"]["b"]),
                  _r2(eeb["ln"]["g"]), _r2(eeb["ln"]["b"]), B_EDGE)

    senders = edge_index[0]
    receivers = edge_index[1]
    zeros = jnp.zeros((N_NODES, H), jnp.float32)

    for k, blk in enumerate(blocks):
        eb, nb = blk["eb"], blk["nb"]
        last = k == len(blocks) - 1
        we = splits[k][2 * H:]
        gsum = _gather_sum(hs, hr, senders, receivers)
        eargs = (gsum, e, we, _r2(eb["l0"]["b"]),
                 eb["l1"]["W"], _r2(eb["l1"]["b"]),
                 eb["l2"]["W"], _r2(eb["l2"]["b"]),
                 _r2(eb["ln"]["g"]), _r2(eb["ln"]["b"]))
        if last:
            e_new = _edge_last_call(*eargs)
        else:
            e_new, e = _edge_call(*eargs)
        a0, a1 = _scatter_add(e_new, receivers, zeros)
        n0 = nb["l0"]["W"]
        nargs = (h, a0, a1, n0[:H], n0[H:], _r2(nb["l0"]["b"]),
                 nb["l1"]["W"], _r2(nb["l1"]["b"]),
                 nb["l2"]["W"], _r2(nb["l2"]["b"]),
                 _r2(nb["ln"]["g"]), _r2(nb["ln"]["b"]))
        if last:
            h = _node_call(*nargs)
        else:
            h, hs, hr = _node_pre_call(*nargs, splits[k + 1][:H],
                                       splits[k + 1][H:2 * H])

    dec = params["dec"]
    on = params["out_norm"]
    frames_p = jnp.pad(frames, ((0, 0), (0, 6)))
    w2p = jnp.pad(dec["l2"]["W"], ((0, 0), (0, 6)))
    b2p = jnp.pad(dec["l2"]["b"], (0, 6))
    stdp = jnp.pad(on["std"], (0, 6), constant_values=1.0)
    meanp = jnp.pad(on["mean"], (0, 6))
    out = _dec_call(h, frames_p,
                    dec["l0"]["W"], _r2(dec["l0"]["b"]),
                    dec["l1"]["W"], _r2(dec["l1"]["b"]),
                    w2p, _r2(b2p), _r2(stdp), _r2(meanp))
    return out[:, :2]
